# Initial kernel scaffold; baseline (speedup 1.0000x reference)
import functools

import jax
import jax.numpy as jnp
from jax import lax
from jax.experimental import pallas as pl
from jax.experimental.pallas import tpu as pltpu
from jax.experimental.pallas import tpu_sc as plsc

NC = 2
NS = 16
L = 16

F32 = jnp.float32
I32 = jnp.int32


def _rsqrt_newton(x):
  bits = lax.bitcast_convert_type(x, I32)
  seed = lax.bitcast_convert_type(jnp.int32(0x5F3759DF) - (bits >> 1), F32)
  xh = x * 0.5
  y = seed
  for _ in range(3):
    y = y * (1.5 - xh * y * y)
  return y


def _zero_fill(ref, nwords):
  z = jnp.zeros((L,), F32)

  def body(i, _):
    ref[pl.ds(i * L, L)] = z
    return 0

  lax.fori_loop(0, nwords // L, body, 0)


def _zero_fill2(ref, nrows, ncols):
  z = jnp.zeros((L,), F32)
  nslice = ncols // L

  def body(i, _):
    for k in range(nslice):
      ref[i, pl.ds(k * L, L)] = z
    return 0

  lax.fori_loop(0, nrows, body, 0)


def _lane_bcast(v, lane):
  return lax.squeeze(lax.slice(v, (lane,), (lane + 1,)), (0,))


def _scale_rows(rows_ref, norm16, j, nslice):
  for lane in range(L):
    e = j * L + lane
    s = _lane_bcast(norm16, lane)
    for k in range(nslice):
      sl = pl.ds(k * L, L)
      rows_ref[e, sl] = rows_ref[e, sl] * s


def _sc_layer1(n, e, d, c, row, col, w, y1):
  ept = e // (NC * NS)
  nchunk = ept // c
  epc = e // NS
  dchunk = epc // c
  rows_per_tile = n // NS
  nslice = d // L

  mesh = plsc.VectorSubcoreMesh(core_axis_name="c", subcore_axis_name="s")

  @functools.partial(
      pl.kernel,
      out_type=(
          jax.ShapeDtypeStruct((e,), F32),
          jax.ShapeDtypeStruct((2 * n, d), F32),
      ),
      mesh=mesh,
      scratch_types=dict(
          deg_sh=pltpu.VMEM_SHARED((n,), F32),
          acc_sh=pltpu.VMEM_SHARED((n, d), F32),
          dinv_v=pltpu.VMEM((n,), F32),
          degb=pltpu.VMEM((n,), F32),
          rows_v=pltpu.VMEM((c, d), F32),
          rowb=pltpu.VMEM((c,), I32),
          colb=pltpu.VMEM((c,), I32),
          wb=pltpu.VMEM((c,), F32),
          normb=pltpu.VMEM((c,), F32),
      ),
  )
  def k(row_h, col_h, w_h, y1_h, norm_h, s1_h, *, deg_sh, acc_sh, dinv_v,
        degb, rows_v, rowb, colb, wb, normb):
    cid = lax.axis_index("c")
    sid = lax.axis_index("s")
    gid = cid * NS + sid

    _zero_fill(dinv_v, n)
    _zero_fill2(rows_v, c, d)

    @pl.when(sid < 10)
    def _():
      pltpu.sync_copy(dinv_v.at[pl.ds(0, n // 10)],
                      deg_sh.at[pl.ds(sid * (n // 10), n // 10)])

    r0 = sid * rows_per_tile
    nfull = rows_per_tile // c
    rem = rows_per_tile - nfull * c
    for b in range(nfull):
      pltpu.sync_copy(rows_v, acc_sh.at[pl.ds(r0 + b * c, c)])
    if rem:
      pltpu.sync_copy(rows_v.at[pl.ds(0, rem)],
                      acc_sh.at[pl.ds(r0 + nfull * c, rem)])
    plsc.subcore_barrier()

    def deg_body(i, _):
      base = sid * epc + i * c
      pltpu.sync_copy(row_h.at[pl.ds(base, c)], rowb)
      pltpu.sync_copy(col_h.at[pl.ds(base, c)], colb)
      pltpu.sync_copy(w_h.at[pl.ds(base, c)], wb)
      for j in range(c // L):
        sl = pl.ds(j * L, L)
        rv, cv, wv = rowb[sl], colb[sl], wb[sl]
        wb[sl] = jnp.where(rv == cv, 0.0, wv)
      pltpu.sync_copy(wb, deg_sh.at[rowb], add=True)
      return 0

    lax.fori_loop(0, dchunk, deg_body, 0)
    plsc.subcore_barrier()

    pltpu.sync_copy(deg_sh, degb)

    def dinv_body(i, _):
      sl = pl.ds(i * L, L)
      dv = degb[sl]
      dinv_v[sl] = jnp.where(dv > 0.0, _rsqrt_newton(jnp.maximum(dv, 1e-30)),
                             0.0)
      return 0

    lax.fori_loop(0, n // L, dinv_body, 0)

    def seg_body(i, _):
      base = gid * ept + i * c
      pltpu.sync_copy(row_h.at[pl.ds(base, c)], rowb)
      pltpu.sync_copy(col_h.at[pl.ds(base, c)], colb)
      pltpu.sync_copy(w_h.at[pl.ds(base, c)], wb)
      for j in range(c // L):
        sl = pl.ds(j * L, L)
        rv, cv, wv = rowb[sl], colb[sl], wb[sl]
        dr = plsc.load_gather(dinv_v, [rv])
        dc = plsc.load_gather(dinv_v, [cv])
        weff = jnp.where(rv == cv, 0.0, wv)
        normb[sl] = -(dr * weff * dc)
      pltpu.sync_copy(normb, norm_h.at[pl.ds(base, c)])
      pltpu.sync_copy(y1_h.at[rowb], rows_v)
      for j in range(c // L):
        _scale_rows(rows_v, normb[pl.ds(j * L, L)], j, nslice)
      pltpu.sync_copy(rows_v, acc_sh.at[colb], add=True)
      return 0

    lax.fori_loop(0, nchunk, seg_body, 0)
    plsc.subcore_barrier()

    pltpu.sync_copy(acc_sh.at[pl.ds(r0, rows_per_tile)],
                    s1_h.at[pl.ds(cid * n + r0, rows_per_tile)])

  return k(row, col, w, y1)


def _sc_layer2(n, e, d, c, row, col, norm, y2):
  ept = e // (NC * NS)
  nchunk = ept // c
  rows_per_tile = n // NS
  nslice = d // L

  mesh = plsc.VectorSubcoreMesh(core_axis_name="c", subcore_axis_name="s")

  @functools.partial(
      pl.kernel,
      out_type=jax.ShapeDtypeStruct((2 * n, d), F32),
      mesh=mesh,
      scratch_types=dict(
          acc_sh=pltpu.VMEM_SHARED((n, d), F32),
          rows_v=pltpu.VMEM((c, d), F32),
          rowb=pltpu.VMEM((c,), I32),
          colb=pltpu.VMEM((c,), I32),
          normb=pltpu.VMEM((c,), F32),
      ),
  )
  def k(row_h, col_h, norm_h, y2_h, s2_h, *, acc_sh, rows_v, rowb, colb,
        normb):
    cid = lax.axis_index("c")
    sid = lax.axis_index("s")
    gid = cid * NS + sid

    _zero_fill2(rows_v, c, d)
    r0 = sid * rows_per_tile
    nfull = rows_per_tile // c
    rem = rows_per_tile - nfull * c
    for b in range(nfull):
      pltpu.sync_copy(rows_v, acc_sh.at[pl.ds(r0 + b * c, c)])
    if rem:
      pltpu.sync_copy(rows_v.at[pl.ds(0, rem)],
                      acc_sh.at[pl.ds(r0 + nfull * c, rem)])
    plsc.subcore_barrier()

    def seg_body(i, _):
      base = gid * ept + i * c
      pltpu.sync_copy(row_h.at[pl.ds(base, c)], rowb)
      pltpu.sync_copy(col_h.at[pl.ds(base, c)], colb)
      pltpu.sync_copy(norm_h.at[pl.ds(base, c)], normb)
      pltpu.sync_copy(y2_h.at[rowb], rows_v)
      for j in range(c // L):
        _scale_rows(rows_v, normb[pl.ds(j * L, L)], j, nslice)
      pltpu.sync_copy(rows_v, acc_sh.at[colb], add=True)
      return 0

    lax.fori_loop(0, nchunk, seg_body, 0)
    plsc.subcore_barrier()

    pltpu.sync_copy(acc_sh.at[pl.ds(r0, rows_per_tile)],
                    s2_h.at[pl.ds(cid * n + r0, rows_per_tile)])

  return k(row, col, norm, y2)


def _tc_matmul(x, wt, bn):
  n, kdim = x.shape
  m = wt.shape[1]

  def body(x_ref, w_ref, o_ref):
    o_ref[...] = jnp.dot(x_ref[...], w_ref[...],
                         preferred_element_type=F32)

  return pl.pallas_call(
      body,
      grid=(n // bn,),
      in_specs=[
          pl.BlockSpec((bn, kdim), lambda i: (i, 0)),
          pl.BlockSpec((kdim, m), lambda i: (0, 0)),
      ],
      out_specs=pl.BlockSpec((bn, m), lambda i: (i, 0)),
      out_shape=jax.ShapeDtypeStruct((n, m), F32),
  )(x, wt)


def _tc_mid(t1, s1, b1, w2t, bn):
  n = t1.shape[0]
  hid = s1.shape[1]
  m = w2t.shape[1]

  def body(t1_ref, s1a_ref, s1b_ref, b1_ref, w_ref, o_ref):
    h = t1_ref[:, hid:] + s1a_ref[...] + s1b_ref[...] + b1_ref[...]
    h = jnp.maximum(h, 0.0)
    o_ref[...] = jnp.dot(h, w_ref[...], preferred_element_type=F32)

  return pl.pallas_call(
      body,
      grid=(n // bn,),
      in_specs=[
          pl.BlockSpec((bn, 2 * hid), lambda i: (i, 0)),
          pl.BlockSpec((bn, hid), lambda i: (i, 0)),
          pl.BlockSpec((bn, hid), lambda i, n_blk=n // bn: (i + n_blk, 0)),
          pl.BlockSpec((1, hid), lambda i: (0, 0)),
          pl.BlockSpec((hid, m), lambda i: (0, 0)),
      ],
      out_specs=pl.BlockSpec((bn, m), lambda i: (i, 0)),
      out_shape=jax.ShapeDtypeStruct((n, m), F32),
  )(t1, s1, s1, b1.reshape(1, hid), w2t)


def _tc_final(hw0, s2, b2, bn):
  n, ncls = hw0.shape
  d2 = s2.shape[1]

  def body(h_ref, s2a_ref, s2b_ref, b2_ref, o_ref):
    o_ref[...] = (h_ref[...] + s2a_ref[:, :ncls] + s2b_ref[:, :ncls]
                  + b2_ref[...])

  return pl.pallas_call(
      body,
      grid=(n // bn,),
      in_specs=[
          pl.BlockSpec((bn, ncls), lambda i: (i, 0)),
          pl.BlockSpec((bn, d2), lambda i: (i, 0)),
          pl.BlockSpec((bn, d2), lambda i, n_blk=n // bn: (i + n_blk, 0)),
          pl.BlockSpec((1, ncls), lambda i: (0, 0)),
      ],
      out_specs=pl.BlockSpec((bn, ncls), lambda i: (i, 0)),
      out_shape=jax.ShapeDtypeStruct((n, ncls), F32),
  )(hw0, s2, s2, b2.reshape(1, ncls))


@jax.jit
def kernel(x, edge_index, edge_weight, W0_1, W1_1, b1, W0_2, W1_2, b2):
  n, _ = x.shape
  e = edge_index.shape[1]
  hid = W0_1.shape[0]
  ncls = W0_2.shape[0]
  d2 = 48
  c = 80
  bn = 1000

  row = edge_index[0]
  col = edge_index[1]

  wc = jnp.concatenate([W1_1, W0_1], axis=0).T
  t1 = _tc_matmul(x, wc, bn)
  y1 = t1[:, :hid]

  norm, s1 = _sc_layer1(n, e, hid, c, row, col, edge_weight, y1)

  w2c = jnp.zeros((hid, d2 + ncls), F32)
  w2c = w2c.at[:, :ncls].set(W1_2.T)
  w2c = w2c.at[:, d2:].set(W0_2.T)
  t2 = _tc_mid(t1, s1, b1, w2c, bn)
  y2 = t2[:, :d2]
  hw0 = t2[:, d2:]

  s2 = _sc_layer2(n, e, d2, c, row, col, norm, y2)

  return _tc_final(hw0, s2, b2, bn)


# speedup vs baseline: 7.4554x; 7.4554x over previous
import functools

import jax
import jax.numpy as jnp
from jax import lax
from jax.experimental import pallas as pl
from jax.experimental.pallas import tpu as pltpu
from jax.experimental.pallas import tpu_sc as plsc

NC = 2
NS = 16
L = 16

F32 = jnp.float32
I32 = jnp.int32


def _rsqrt_newton(x):
  bits = lax.bitcast_convert_type(x, I32)
  seed = lax.bitcast_convert_type(jnp.int32(0x5F3759DF) - (bits >> 1), F32)
  xh = x * 0.5
  y = seed
  for _ in range(3):
    y = y * (1.5 - xh * y * y)
  return y


def _zero_fill(ref, nwords):
  z = jnp.zeros((L,), F32)

  def body(i, _):
    ref[pl.ds(i * L, L)] = z
    return 0

  lax.fori_loop(0, nwords // L, body, 0)


def _zero_fill2(ref, nrows, ncols):
  z = jnp.zeros((L,), F32)
  nslice = ncols // L

  def body(i, _):
    for k in range(nslice):
      ref[i, pl.ds(k * L, L)] = z
    return 0

  lax.fori_loop(0, nrows, body, 0)


def _lane_bcast(v, lane):
  return lax.squeeze(lax.slice(v, (lane,), (lane + 1,)), (0,))


def _scale_rows(rows_ref, norm16, j, nslice):
  for lane in range(L):
    e = j * L + lane
    s = _lane_bcast(norm16, lane)
    for k in range(nslice):
      sl = pl.ds(k * L, L)
      rows_ref[e, sl] = rows_ref[e, sl] * s


def _sc_layer1(n, e, d, c, row, col, w, y1):
  ept = e // (NC * NS)
  nchunk = ept // c
  epc = e // NS
  dchunk = epc // c
  rslice = 1000
  ntiles_io = n // rslice
  nslice = d // L

  mesh = plsc.VectorSubcoreMesh(core_axis_name="c", subcore_axis_name="s")

  @functools.partial(
      pl.kernel,
      out_type=(
          jax.ShapeDtypeStruct((e,), F32),
          jax.ShapeDtypeStruct((2 * n, d), F32),
      ),
      mesh=mesh,
      compiler_params=pltpu.CompilerParams(needs_layout_passes=False, use_tc_tiling_on_sc=False),
      scratch_types=dict(
          deg_sh=pltpu.VMEM_SHARED((n,), F32),
          acc_sh=pltpu.VMEM_SHARED((n, d), F32),
          dinv_v=pltpu.VMEM((n,), F32),
          degb=pltpu.VMEM((n,), F32),
          rows_v=pltpu.VMEM((c, d), F32),
          rowb=pltpu.VMEM((c,), I32),
          colb=pltpu.VMEM((c,), I32),
          wb=pltpu.VMEM((c,), F32),
          normb=pltpu.VMEM((c,), F32),
      ),
  )
  def k(row_h, col_h, w_h, y1_h, norm_h, s1_h, *, deg_sh, acc_sh, dinv_v,
        degb, rows_v, rowb, colb, wb, normb):
    cid = lax.axis_index("c")
    sid = lax.axis_index("s")
    gid = cid * NS + sid

    _zero_fill(dinv_v, n)
    _zero_fill2(rows_v, c, d)

    r0 = sid * rslice
    nfull = rslice // c
    rem = rslice - nfull * c

    @pl.when(sid < ntiles_io)
    def _():
      pltpu.sync_copy(dinv_v.at[pl.ds(0, rslice)],
                      deg_sh.at[pl.ds(r0, rslice)])
      for b in range(nfull):
        pltpu.sync_copy(rows_v, acc_sh.at[pl.ds(r0 + b * c, c)])
      if rem:
        pltpu.sync_copy(rows_v.at[pl.ds(0, rem)],
                        acc_sh.at[pl.ds(r0 + nfull * c, rem)])

    plsc.subcore_barrier()

    def deg_body(i, _):
      base = sid * epc + i * c
      pltpu.sync_copy(row_h.at[pl.ds(base, c)], rowb)
      pltpu.sync_copy(col_h.at[pl.ds(base, c)], colb)
      pltpu.sync_copy(w_h.at[pl.ds(base, c)], wb)
      for j in range(c // L):
        sl = pl.ds(j * L, L)
        rv, cv, wv = rowb[sl], colb[sl], wb[sl]
        wb[sl] = jnp.where(rv == cv, 0.0, wv)
      pltpu.sync_copy(wb, deg_sh.at[rowb], add=True)
      return 0

    lax.fori_loop(0, dchunk, deg_body, 0)
    plsc.subcore_barrier()

    pltpu.sync_copy(deg_sh, degb)

    def dinv_body(i, _):
      sl = pl.ds(i * L, L)
      dv = degb[sl]
      dinv_v[sl] = jnp.where(dv > 0.0, _rsqrt_newton(jnp.maximum(dv, 1e-30)),
                             0.0)
      return 0

    lax.fori_loop(0, n // L, dinv_body, 0)

    def seg_body(i, _):
      base = gid * ept + i * c
      pltpu.sync_copy(row_h.at[pl.ds(base, c)], rowb)
      pltpu.sync_copy(col_h.at[pl.ds(base, c)], colb)
      pltpu.sync_copy(w_h.at[pl.ds(base, c)], wb)
      for j in range(c // L):
        sl = pl.ds(j * L, L)
        rv, cv, wv = rowb[sl], colb[sl], wb[sl]
        dr = plsc.load_gather(dinv_v, [rv])
        dc = plsc.load_gather(dinv_v, [cv])
        weff = jnp.where(rv == cv, 0.0, wv)
        normb[sl] = -(dr * weff * dc)
      pltpu.sync_copy(normb, norm_h.at[pl.ds(base, c)])
      pltpu.sync_copy(y1_h.at[rowb], rows_v)
      for j in range(c // L):
        _scale_rows(rows_v, normb[pl.ds(j * L, L)], j, nslice)
      pltpu.sync_copy(rows_v, acc_sh.at[colb], add=True)
      return 0

    lax.fori_loop(0, nchunk, seg_body, 0)
    plsc.subcore_barrier()

    @pl.when(sid < ntiles_io)
    def _():
      pltpu.sync_copy(acc_sh.at[pl.ds(r0, rslice)],
                      s1_h.at[pl.ds(cid * n + r0, rslice)])

  return k(row, col, w, y1)


def _sc_layer2(n, e, d, c, row, col, norm, y2):
  ept = e // (NC * NS)
  nchunk = ept // c
  rslice = 1000
  ntiles_io = n // rslice
  nslice = d // L

  mesh = plsc.VectorSubcoreMesh(core_axis_name="c", subcore_axis_name="s")

  @functools.partial(
      pl.kernel,
      out_type=jax.ShapeDtypeStruct((2 * n, d), F32),
      mesh=mesh,
      compiler_params=pltpu.CompilerParams(needs_layout_passes=False, use_tc_tiling_on_sc=False),
      scratch_types=dict(
          acc_sh=pltpu.VMEM_SHARED((n, d), F32),
          rows_v=pltpu.VMEM((c, d), F32),
          rowb=pltpu.VMEM((c,), I32),
          colb=pltpu.VMEM((c,), I32),
          normb=pltpu.VMEM((c,), F32),
      ),
  )
  def k(row_h, col_h, norm_h, y2_h, s2_h, *, acc_sh, rows_v, rowb, colb,
        normb):
    cid = lax.axis_index("c")
    sid = lax.axis_index("s")
    gid = cid * NS + sid

    _zero_fill2(rows_v, c, d)
    r0 = sid * rslice
    nfull = rslice // c
    rem = rslice - nfull * c

    @pl.when(sid < ntiles_io)
    def _():
      for b in range(nfull):
        pltpu.sync_copy(rows_v, acc_sh.at[pl.ds(r0 + b * c, c)])
      if rem:
        pltpu.sync_copy(rows_v.at[pl.ds(0, rem)],
                        acc_sh.at[pl.ds(r0 + nfull * c, rem)])

    plsc.subcore_barrier()

    def seg_body(i, _):
      base = gid * ept + i * c
      pltpu.sync_copy(row_h.at[pl.ds(base, c)], rowb)
      pltpu.sync_copy(col_h.at[pl.ds(base, c)], colb)
      pltpu.sync_copy(norm_h.at[pl.ds(base, c)], normb)
      pltpu.sync_copy(y2_h.at[rowb], rows_v)
      for j in range(c // L):
        _scale_rows(rows_v, normb[pl.ds(j * L, L)], j, nslice)
      pltpu.sync_copy(rows_v, acc_sh.at[colb], add=True)
      return 0

    lax.fori_loop(0, nchunk, seg_body, 0)
    plsc.subcore_barrier()

    @pl.when(sid < ntiles_io)
    def _():
      pltpu.sync_copy(acc_sh.at[pl.ds(r0, rslice)],
                      s2_h.at[pl.ds(cid * n + r0, rslice)])

  return k(row, col, norm, y2)


def _tc_matmul(x, wt, bn):
  n, kdim = x.shape
  m = wt.shape[1]

  def body(x_ref, w_ref, o_ref):
    o_ref[...] = jnp.dot(x_ref[...], w_ref[...],
                         preferred_element_type=F32)

  return pl.pallas_call(
      body,
      grid=(n // bn,),
      in_specs=[
          pl.BlockSpec((bn, kdim), lambda i: (i, 0)),
          pl.BlockSpec((kdim, m), lambda i: (0, 0)),
      ],
      out_specs=pl.BlockSpec((bn, m), lambda i: (i, 0)),
      out_shape=jax.ShapeDtypeStruct((n, m), F32),
  )(x, wt)


def _tc_mid(t1, s1, b1, w2t, bn):
  n = t1.shape[0]
  hid = s1.shape[1]
  m = w2t.shape[1]

  def body(t1_ref, s1a_ref, s1b_ref, b1_ref, w_ref, o_ref):
    h = t1_ref[:, hid:] + s1a_ref[...] + s1b_ref[...] + b1_ref[...]
    h = jnp.maximum(h, 0.0)
    o_ref[...] = jnp.dot(h, w_ref[...], preferred_element_type=F32)

  return pl.pallas_call(
      body,
      grid=(n // bn,),
      in_specs=[
          pl.BlockSpec((bn, 2 * hid), lambda i: (i, 0)),
          pl.BlockSpec((bn, hid), lambda i: (i, 0)),
          pl.BlockSpec((bn, hid), lambda i, n_blk=n // bn: (i + n_blk, 0)),
          pl.BlockSpec((1, hid), lambda i: (0, 0)),
          pl.BlockSpec((hid, m), lambda i: (0, 0)),
      ],
      out_specs=pl.BlockSpec((bn, m), lambda i: (i, 0)),
      out_shape=jax.ShapeDtypeStruct((n, m), F32),
  )(t1, s1, s1, b1.reshape(1, hid), w2t)


def _tc_final(hw0, s2, b2, bn):
  n, ncls = hw0.shape
  d2 = s2.shape[1]

  def body(h_ref, s2a_ref, s2b_ref, b2_ref, o_ref):
    o_ref[...] = (h_ref[...] + s2a_ref[:, :ncls] + s2b_ref[:, :ncls]
                  + b2_ref[...])

  return pl.pallas_call(
      body,
      grid=(n // bn,),
      in_specs=[
          pl.BlockSpec((bn, ncls), lambda i: (i, 0)),
          pl.BlockSpec((bn, d2), lambda i: (i, 0)),
          pl.BlockSpec((bn, d2), lambda i, n_blk=n // bn: (i + n_blk, 0)),
          pl.BlockSpec((1, ncls), lambda i: (0, 0)),
      ],
      out_specs=pl.BlockSpec((bn, ncls), lambda i: (i, 0)),
      out_shape=jax.ShapeDtypeStruct((n, ncls), F32),
  )(hw0, s2, s2, b2.reshape(1, ncls))


@jax.jit
def kernel(x, edge_index, edge_weight, W0_1, W1_1, b1, W0_2, W1_2, b2):
  n, _ = x.shape
  e = edge_index.shape[1]
  hid = W0_1.shape[0]
  ncls = W0_2.shape[0]
  d2 = 48
  c = 80
  bn = 1000

  row = edge_index[0]
  col = edge_index[1]

  wc = jnp.concatenate([W1_1, W0_1], axis=0).T
  t1 = _tc_matmul(x, wc, bn)
  y1 = t1[:, :hid]

  norm, s1 = _sc_layer1(n, e, hid, c, row, col, edge_weight, y1)

  w2c = jnp.zeros((hid, d2 + ncls), F32)
  w2c = w2c.at[:, :ncls].set(W1_2.T)
  w2c = w2c.at[:, d2:].set(W0_2.T)
  t2 = _tc_mid(t1, s1, b1, w2c, bn)
  y2 = t2[:, :d2]
  hw0 = t2[:, d2:]

  s2 = _sc_layer2(n, e, d2, c, row, col, norm, y2)

  return _tc_final(hw0, s2, b2, bn)


# speedup vs baseline: 16.7945x; 2.2527x over previous
import functools

import jax
import jax.numpy as jnp
from jax import lax
from jax.experimental import pallas as pl
from jax.experimental.pallas import tpu as pltpu
from jax.experimental.pallas import tpu_sc as plsc

NC = 2
NS = 16
L = 16

F32 = jnp.float32
I32 = jnp.int32


def _rsqrt_newton(x):
  bits = lax.bitcast_convert_type(x, I32)
  seed = lax.bitcast_convert_type(jnp.int32(0x5F3759DF) - (bits >> 1), F32)
  xh = x * 0.5
  y = seed
  for _ in range(3):
    y = y * (1.5 - xh * y * y)
  return y


def _zero_fill(ref, nwords):
  z = jnp.zeros((L,), F32)

  def body(i, _):
    ref[pl.ds(i * L, L)] = z
    return 0

  lax.fori_loop(0, nwords // L, body, 0)


def _zero_fill2(ref, nrows, ncols):
  z = jnp.zeros((L,), F32)
  nslice = ncols // L

  def body(i, _):
    for k in range(nslice):
      ref[i, pl.ds(k * L, L)] = z
    return 0

  lax.fori_loop(0, nrows, body, 0)


def _lane_bcast(v, lane):
  return lax.squeeze(lax.slice(v, (lane,), (lane + 1,)), (0,))


def _scale_rows(rows_ref, norm16, j, nslice):
  for lane in range(L):
    e = j * L + lane
    s = _lane_bcast(norm16, lane)
    for k in range(nslice):
      sl = pl.ds(k * L, L)
      rows_ref[e, sl] = rows_ref[e, sl] * s


def _sc_layer1(n, e, d, c, row, col, w, y1):
  ept = e // (NC * NS)
  nchunk = ept // c
  rslice = 1000
  ntiles_io = n // rslice
  nslice = d // L

  mesh = plsc.VectorSubcoreMesh(core_axis_name="c", subcore_axis_name="s")

  @functools.partial(
      pl.kernel,
      out_type=(
          jax.ShapeDtypeStruct((e,), F32),
          jax.ShapeDtypeStruct((2 * n, d), F32),
      ),
      mesh=mesh,
      compiler_params=pltpu.CompilerParams(needs_layout_passes=False, use_tc_tiling_on_sc=False),
      scratch_types=dict(
          deg_sh=pltpu.VMEM_SHARED((n,), F32),
          acc_sh=pltpu.VMEM_SHARED((n, d), F32),
          dinv_v=pltpu.VMEM((n,), F32),
          rowT=pltpu.VMEM((ept,), I32),
          colT=pltpu.VMEM((ept,), I32),
          wT=pltpu.VMEM((ept,), F32),
          rowTo=pltpu.VMEM((ept,), I32),
          colTo=pltpu.VMEM((ept,), I32),
          wTo=pltpu.VMEM((ept,), F32),
          normT=pltpu.VMEM((ept,), F32),
          rows_v=pltpu.VMEM((c, d), F32),
          rowb=pltpu.VMEM((c,), I32),
          colb=pltpu.VMEM((c,), I32),
          wb=pltpu.VMEM((c,), F32),
      ),
  )
  def k(row_h, col_h, w_h, y1_h, norm_h, s1_h, *, deg_sh, acc_sh, dinv_v,
        rowT, colT, wT, rowTo, colTo, wTo, normT, rows_v, rowb, colb, wb):
    cid = lax.axis_index("c")
    sid = lax.axis_index("s")
    gid = cid * NS + sid
    obid = (1 - cid) * NS + sid

    pltpu.sync_copy(row_h.at[pl.ds(gid * ept, ept)], rowT)
    pltpu.sync_copy(col_h.at[pl.ds(gid * ept, ept)], colT)
    pltpu.sync_copy(w_h.at[pl.ds(gid * ept, ept)], wT)
    pltpu.sync_copy(row_h.at[pl.ds(obid * ept, ept)], rowTo)
    pltpu.sync_copy(col_h.at[pl.ds(obid * ept, ept)], colTo)
    pltpu.sync_copy(w_h.at[pl.ds(obid * ept, ept)], wTo)

    _zero_fill(dinv_v, n)
    _zero_fill2(rows_v, c, d)

    r0 = sid * rslice
    nfull = rslice // c
    rem = rslice - nfull * c

    @pl.when(sid < ntiles_io)
    def _():
      pltpu.sync_copy(dinv_v.at[pl.ds(0, rslice)],
                      deg_sh.at[pl.ds(r0, rslice)])
      for b in range(nfull):
        pltpu.sync_copy(rows_v, acc_sh.at[pl.ds(r0 + b * c, c)])
      if rem:
        pltpu.sync_copy(rows_v.at[pl.ds(0, rem)],
                        acc_sh.at[pl.ds(r0 + nfull * c, rem)])

    plsc.subcore_barrier()

    for (rT_, cT_, wT_) in ((rowT, colT, wT), (rowTo, colTo, wTo)):
      def deg_body(i, _, rT_=rT_, cT_=cT_, wT_=wT_):
        base = i * c
        for j in range(c // L):
          src = pl.ds(base + j * L, L)
          dst = pl.ds(j * L, L)
          rv, cv, wv = rT_[src], cT_[src], wT_[src]
          rowb[dst] = rv
          wb[dst] = jnp.where(rv == cv, 0.0, wv)
        pltpu.sync_copy(wb, deg_sh.at[rowb], add=True)
        return 0

      lax.fori_loop(0, nchunk, deg_body, 0)
    plsc.subcore_barrier()

    pltpu.sync_copy(deg_sh, dinv_v)

    def dinv_body(i, _):
      sl = pl.ds(i * L, L)
      dv = dinv_v[sl]
      dinv_v[sl] = jnp.where(dv > 0.0, _rsqrt_newton(jnp.maximum(dv, 1e-30)),
                             0.0)
      return 0

    lax.fori_loop(0, n // L, dinv_body, 0)

    def seg_body(i, _):
      base = i * c
      pltpu.sync_copy(y1_h.at[rowT.at[pl.ds(base, c)]], rows_v)
      for j in range(c // L):
        src = pl.ds(base + j * L, L)
        rv, cv, wv = rowT[src], colT[src], wT[src]
        dr = plsc.load_gather(dinv_v, [rv])
        dc = plsc.load_gather(dinv_v, [cv])
        weff = jnp.where(rv == cv, 0.0, wv)
        normT[src] = -(dr * weff * dc)
        colb[pl.ds(j * L, L)] = cv
      for j in range(c // L):
        _scale_rows(rows_v, normT[pl.ds(base + j * L, L)], j, nslice)
      pltpu.sync_copy(rows_v, acc_sh.at[colb], add=True)
      return 0

    lax.fori_loop(0, nchunk, seg_body, 0)
    pltpu.sync_copy(normT, norm_h.at[pl.ds(gid * ept, ept)])
    plsc.subcore_barrier()

    @pl.when(sid < ntiles_io)
    def _():
      pltpu.sync_copy(acc_sh.at[pl.ds(r0, rslice)],
                      s1_h.at[pl.ds(cid * n + r0, rslice)])

  return k(row, col, w, y1)


def _sc_layer2(n, e, d, c, row, col, norm, y2):
  ept = e // (NC * NS)
  nchunk = ept // c
  rslice = 1000
  ntiles_io = n // rslice
  nslice = d // L

  mesh = plsc.VectorSubcoreMesh(core_axis_name="c", subcore_axis_name="s")

  @functools.partial(
      pl.kernel,
      out_type=jax.ShapeDtypeStruct((2 * n, d), F32),
      mesh=mesh,
      compiler_params=pltpu.CompilerParams(needs_layout_passes=False, use_tc_tiling_on_sc=False),
      scratch_types=dict(
          acc_sh=pltpu.VMEM_SHARED((n, d), F32),
          rowT=pltpu.VMEM((ept,), I32),
          colT=pltpu.VMEM((ept,), I32),
          normT=pltpu.VMEM((ept,), F32),
          rows_v=pltpu.VMEM((c, d), F32),
          colb=pltpu.VMEM((c,), I32),
      ),
  )
  def k(row_h, col_h, norm_h, y2_h, s2_h, *, acc_sh, rowT, colT, normT,
        rows_v, colb):
    cid = lax.axis_index("c")
    sid = lax.axis_index("s")
    gid = cid * NS + sid

    pltpu.sync_copy(row_h.at[pl.ds(gid * ept, ept)], rowT)
    pltpu.sync_copy(col_h.at[pl.ds(gid * ept, ept)], colT)
    pltpu.sync_copy(norm_h.at[pl.ds(gid * ept, ept)], normT)

    _zero_fill2(rows_v, c, d)
    r0 = sid * rslice
    nfull = rslice // c
    rem = rslice - nfull * c

    @pl.when(sid < ntiles_io)
    def _():
      for b in range(nfull):
        pltpu.sync_copy(rows_v, acc_sh.at[pl.ds(r0 + b * c, c)])
      if rem:
        pltpu.sync_copy(rows_v.at[pl.ds(0, rem)],
                        acc_sh.at[pl.ds(r0 + nfull * c, rem)])

    plsc.subcore_barrier()

    def seg_body(i, _):
      base = i * c
      pltpu.sync_copy(y2_h.at[rowT.at[pl.ds(base, c)]], rows_v)
      for j in range(c // L):
        colb[pl.ds(j * L, L)] = colT[pl.ds(base + j * L, L)]
      for j in range(c // L):
        _scale_rows(rows_v, normT[pl.ds(base + j * L, L)], j, nslice)
      pltpu.sync_copy(rows_v, acc_sh.at[colb], add=True)
      return 0

    lax.fori_loop(0, nchunk, seg_body, 0)
    plsc.subcore_barrier()

    @pl.when(sid < ntiles_io)
    def _():
      pltpu.sync_copy(acc_sh.at[pl.ds(r0, rslice)],
                      s2_h.at[pl.ds(cid * n + r0, rslice)])

  return k(row, col, norm, y2)


def _tc_matmul(x, wt, bn):
  n, kdim = x.shape
  m = wt.shape[1]

  def body(x_ref, w_ref, o_ref):
    o_ref[...] = jnp.dot(x_ref[...], w_ref[...],
                         preferred_element_type=F32)

  return pl.pallas_call(
      body,
      grid=(n // bn,),
      in_specs=[
          pl.BlockSpec((bn, kdim), lambda i: (i, 0)),
          pl.BlockSpec((kdim, m), lambda i: (0, 0)),
      ],
      out_specs=pl.BlockSpec((bn, m), lambda i: (i, 0)),
      out_shape=jax.ShapeDtypeStruct((n, m), F32),
  )(x, wt)


def _tc_mid(t1, s1, b1, w2t, bn):
  n = t1.shape[0]
  hid = s1.shape[1]
  m = w2t.shape[1]

  def body(t1_ref, s1a_ref, s1b_ref, b1_ref, w_ref, o_ref):
    h = t1_ref[:, hid:] + s1a_ref[...] + s1b_ref[...] + b1_ref[...]
    h = jnp.maximum(h, 0.0)
    o_ref[...] = jnp.dot(h, w_ref[...], preferred_element_type=F32)

  return pl.pallas_call(
      body,
      grid=(n // bn,),
      in_specs=[
          pl.BlockSpec((bn, 2 * hid), lambda i: (i, 0)),
          pl.BlockSpec((bn, hid), lambda i: (i, 0)),
          pl.BlockSpec((bn, hid), lambda i, n_blk=n // bn: (i + n_blk, 0)),
          pl.BlockSpec((1, hid), lambda i: (0, 0)),
          pl.BlockSpec((hid, m), lambda i: (0, 0)),
      ],
      out_specs=pl.BlockSpec((bn, m), lambda i: (i, 0)),
      out_shape=jax.ShapeDtypeStruct((n, m), F32),
  )(t1, s1, s1, b1.reshape(1, hid), w2t)


def _tc_final(hw0, s2, b2, bn):
  n, ncls = hw0.shape
  d2 = s2.shape[1]

  def body(h_ref, s2a_ref, s2b_ref, b2_ref, o_ref):
    o_ref[...] = (h_ref[...] + s2a_ref[:, :ncls] + s2b_ref[:, :ncls]
                  + b2_ref[...])

  return pl.pallas_call(
      body,
      grid=(n // bn,),
      in_specs=[
          pl.BlockSpec((bn, ncls), lambda i: (i, 0)),
          pl.BlockSpec((bn, d2), lambda i: (i, 0)),
          pl.BlockSpec((bn, d2), lambda i, n_blk=n // bn: (i + n_blk, 0)),
          pl.BlockSpec((1, ncls), lambda i: (0, 0)),
      ],
      out_specs=pl.BlockSpec((bn, ncls), lambda i: (i, 0)),
      out_shape=jax.ShapeDtypeStruct((n, ncls), F32),
  )(hw0, s2, s2, b2.reshape(1, ncls))


@jax.jit
def kernel(x, edge_index, edge_weight, W0_1, W1_1, b1, W0_2, W1_2, b2):
  n, _ = x.shape
  e = edge_index.shape[1]
  hid = W0_1.shape[0]
  ncls = W0_2.shape[0]
  d2 = 48
  c = 80
  bn = 1000

  row = edge_index[0]
  col = edge_index[1]

  wc = jnp.concatenate([W1_1, W0_1], axis=0).T
  t1 = _tc_matmul(x, wc, bn)
  y1 = t1[:, :hid]

  norm, s1 = _sc_layer1(n, e, hid, c, row, col, edge_weight, y1)

  w2c = jnp.zeros((hid, d2 + ncls), F32)
  w2c = w2c.at[:, :ncls].set(W1_2.T)
  w2c = w2c.at[:, d2:].set(W0_2.T)
  t2 = _tc_mid(t1, s1, b1, w2c, bn)
  y2 = t2[:, :d2]
  hw0 = t2[:, d2:]

  s2 = _sc_layer2(n, e, d2, c, row, col, norm, y2)

  return _tc_final(hw0, s2, b2, bn)


# speedup vs baseline: 21.2811x; 1.2671x over previous
import functools

import jax
import jax.numpy as jnp
from jax import lax
from jax.experimental import pallas as pl
from jax.experimental.pallas import tpu as pltpu
from jax.experimental.pallas import tpu_sc as plsc

NC = 2
NS = 16
L = 16

F32 = jnp.float32
I32 = jnp.int32


def _rsqrt_newton(x):
  bits = lax.bitcast_convert_type(x, I32)
  seed = lax.bitcast_convert_type(jnp.int32(0x5F3759DF) - (bits >> 1), F32)
  xh = x * 0.5
  y = seed
  for _ in range(3):
    y = y * (1.5 - xh * y * y)
  return y


def _zero_fill(ref, nwords):
  z = jnp.zeros((L,), F32)

  def body(i, _):
    ref[pl.ds(i * L, L)] = z
    return 0

  lax.fori_loop(0, nwords // L, body, 0)


def _zero_fill2(ref, nrows, ncols):
  z = jnp.zeros((L,), F32)
  nslice = ncols // L

  def body(i, _):
    for k in range(nslice):
      ref[i, pl.ds(k * L, L)] = z
    return 0

  lax.fori_loop(0, nrows, body, 0)


def _lane_bcast(v, lane):
  return lax.squeeze(lax.slice(v, (lane,), (lane + 1,)), (0,))


def _scale_rows(rows_ref, norm16, j, nslice):
  for lane in range(L):
    e = j * L + lane
    s = _lane_bcast(norm16, lane)
    for k in range(nslice):
      sl = pl.ds(k * L, L)
      rows_ref[e, sl] = rows_ref[e, sl] * s


def _sc_layer1(n, e, d, c, row, col, w, y1):
  ept = e // (NC * NS)
  nchunk = ept // c
  rslice = 1000
  ntiles_io = n // rslice
  nslice = d // L

  mesh = plsc.VectorSubcoreMesh(core_axis_name="c", subcore_axis_name="s")

  @functools.partial(
      pl.kernel,
      out_type=(
          jax.ShapeDtypeStruct((e,), F32),
          jax.ShapeDtypeStruct((2 * n, d), F32),
      ),
      mesh=mesh,
      compiler_params=pltpu.CompilerParams(needs_layout_passes=False, use_tc_tiling_on_sc=False),
      scratch_types=dict(
          deg_sh=pltpu.VMEM_SHARED((n,), F32),
          acc_sh=pltpu.VMEM_SHARED((n, d), F32),
          dinv_v=pltpu.VMEM((n,), F32),
          rowT=pltpu.VMEM((ept,), I32),
          colT=pltpu.VMEM((ept,), I32),
          wT=pltpu.VMEM((ept,), F32),
          rowTo=pltpu.VMEM((ept // 5,), I32),
          colTo=pltpu.VMEM((ept // 5,), I32),
          wTo=pltpu.VMEM((ept // 5,), F32),
          normT=pltpu.VMEM((ept,), F32),
          rows_a=pltpu.VMEM((c, d), F32),
          rows_b=pltpu.VMEM((c, d), F32),
          rowb=pltpu.VMEM((c,), I32),
          colb_a=pltpu.VMEM((c,), I32),
          colb_b=pltpu.VMEM((c,), I32),
          wb=pltpu.VMEM((c,), F32),
          gsem_a=pltpu.SemaphoreType.DMA,
          gsem_b=pltpu.SemaphoreType.DMA,
      ),
  )
  def k(row_h, col_h, w_h, y1_h, norm_h, s1_h, *, deg_sh, acc_sh, dinv_v,
        rowT, colT, wT, rowTo, colTo, wTo, normT, rows_a, rows_b,
        rowb, colb_a, colb_b, wb, gsem_a, gsem_b):
    cid = lax.axis_index("c")
    sid = lax.axis_index("s")
    gid = cid * NS + sid
    obid = (1 - cid) * NS + sid

    pltpu.sync_copy(row_h.at[pl.ds(gid * ept, ept)], rowT)
    pltpu.sync_copy(col_h.at[pl.ds(gid * ept, ept)], colT)
    pltpu.sync_copy(w_h.at[pl.ds(gid * ept, ept)], wT)

    _zero_fill(dinv_v, n)
    _zero_fill2(rows_a, c, d)

    r0 = sid * rslice
    nfull = rslice // c
    rem = rslice - nfull * c

    @pl.when(sid < ntiles_io)
    def _():
      pltpu.sync_copy(dinv_v.at[pl.ds(0, rslice)],
                      deg_sh.at[pl.ds(r0, rslice)])
      for b in range(nfull):
        pltpu.sync_copy(rows_a, acc_sh.at[pl.ds(r0 + b * c, c)])
      if rem:
        pltpu.sync_copy(rows_a.at[pl.ds(0, rem)],
                        acc_sh.at[pl.ds(r0 + nfull * c, rem)])

    plsc.subcore_barrier()

    def deg_chunks(rT_, cT_, wT_, count):
      def deg_body(i, _):
        base = i * c
        for j in range(c // L):
          srcsl = pl.ds(base + j * L, L)
          dst = pl.ds(j * L, L)
          rv, cv, wv = rT_[srcsl], cT_[srcsl], wT_[srcsl]
          rowb[dst] = rv
          wb[dst] = jnp.where(rv == cv, 0.0, wv)
        pltpu.sync_copy(wb, deg_sh.at[rowb], add=True)
        return 0

      lax.fori_loop(0, count, deg_body, 0)

    deg_chunks(rowT, colT, wT, nchunk)
    piece = ept // 5
    for p in range(5):
      pltpu.sync_copy(row_h.at[pl.ds(obid * ept + p * piece, piece)], rowTo)
      pltpu.sync_copy(col_h.at[pl.ds(obid * ept + p * piece, piece)], colTo)
      pltpu.sync_copy(w_h.at[pl.ds(obid * ept + p * piece, piece)], wTo)
      deg_chunks(rowTo, colTo, wTo, piece // c)
    plsc.subcore_barrier()

    pltpu.sync_copy(deg_sh, dinv_v)

    def dinv_body(i, _):
      sl = pl.ds(i * L, L)
      dv = dinv_v[sl]
      dinv_v[sl] = jnp.where(dv > 0.0, _rsqrt_newton(jnp.maximum(dv, 1e-30)),
                             0.0)
      return 0

    lax.fori_loop(0, n // L, dinv_body, 0)

    slots = ((rows_a, colb_a, gsem_a), (rows_b, colb_b, gsem_b))

    def start_gather(i, slot):
      rows, _, sem = slots[slot]
      pltpu.async_copy(y1_h.at[rowT.at[pl.ds(i * c, c)]], rows, sem)

    def wait_gather(slot):
      rows, _, sem = slots[slot]
      pltpu.make_async_copy(y1_h.at[pl.ds(0, c)], rows, sem).wait()

    def process(i, slot):
      rows, colb, _ = slots[slot]
      base = i * c
      for j in range(c // L):
        src = pl.ds(base + j * L, L)
        rv, cv, wv = rowT[src], colT[src], wT[src]
        dr = plsc.load_gather(dinv_v, [rv])
        dc = plsc.load_gather(dinv_v, [cv])
        weff = jnp.where(rv == cv, 0.0, wv)
        normT[src] = -(dr * weff * dc)
        colb[pl.ds(j * L, L)] = cv
      wait_gather(slot)

      @pl.when(i + 1 < nchunk)
      def _():
        start_gather(i + 1, 1 - slot)

      for j in range(c // L):
        _scale_rows(rows, normT[pl.ds(base + j * L, L)], j, nslice)
      pltpu.sync_copy(rows, acc_sh.at[colb], add=True)

    start_gather(0, 0)

    def seg_body(i2, _):
      for par in range(2):
        i = 2 * i2 + par

        @pl.when(i < nchunk)
        def _():
          process(i, par)

      return 0

    lax.fori_loop(0, (nchunk + 1) // 2, seg_body, 0)
    pltpu.sync_copy(normT, norm_h.at[pl.ds(gid * ept, ept)])
    plsc.subcore_barrier()

    @pl.when(sid < ntiles_io)
    def _():
      pltpu.sync_copy(acc_sh.at[pl.ds(r0, rslice)],
                      s1_h.at[pl.ds(cid * n + r0, rslice)])

  return k(row, col, w, y1)


def _sc_layer2(n, e, d, c, row, col, norm, y2):
  ept = e // (NC * NS)
  nchunk = ept // c
  rslice = 1000
  ntiles_io = n // rslice
  nslice = d // L

  mesh = plsc.VectorSubcoreMesh(core_axis_name="c", subcore_axis_name="s")

  @functools.partial(
      pl.kernel,
      out_type=jax.ShapeDtypeStruct((2 * n, d), F32),
      mesh=mesh,
      compiler_params=pltpu.CompilerParams(needs_layout_passes=False, use_tc_tiling_on_sc=False),
      scratch_types=dict(
          acc_sh=pltpu.VMEM_SHARED((n, d), F32),
          rowT=pltpu.VMEM((ept,), I32),
          colT=pltpu.VMEM((ept,), I32),
          normT=pltpu.VMEM((ept,), F32),
          rows_a=pltpu.VMEM((c, d), F32),
          rows_b=pltpu.VMEM((c, d), F32),
          colb_a=pltpu.VMEM((c,), I32),
          colb_b=pltpu.VMEM((c,), I32),
          gsem_a=pltpu.SemaphoreType.DMA,
          gsem_b=pltpu.SemaphoreType.DMA,
      ),
  )
  def k(row_h, col_h, norm_h, y2_h, s2_h, *, acc_sh, rowT, colT, normT,
        rows_a, rows_b, colb_a, colb_b, gsem_a, gsem_b):
    cid = lax.axis_index("c")
    sid = lax.axis_index("s")
    gid = cid * NS + sid

    pltpu.sync_copy(row_h.at[pl.ds(gid * ept, ept)], rowT)
    pltpu.sync_copy(col_h.at[pl.ds(gid * ept, ept)], colT)
    pltpu.sync_copy(norm_h.at[pl.ds(gid * ept, ept)], normT)

    _zero_fill2(rows_a, c, d)
    r0 = sid * rslice
    nfull = rslice // c
    rem = rslice - nfull * c

    @pl.when(sid < ntiles_io)
    def _():
      for b in range(nfull):
        pltpu.sync_copy(rows_a, acc_sh.at[pl.ds(r0 + b * c, c)])
      if rem:
        pltpu.sync_copy(rows_a.at[pl.ds(0, rem)],
                        acc_sh.at[pl.ds(r0 + nfull * c, rem)])

    plsc.subcore_barrier()

    slots = ((rows_a, colb_a, gsem_a), (rows_b, colb_b, gsem_b))

    def start_gather(i, slot):
      rows, _, sem = slots[slot]
      pltpu.async_copy(y2_h.at[rowT.at[pl.ds(i * c, c)]], rows, sem)

    def wait_gather(slot):
      rows, _, sem = slots[slot]
      pltpu.make_async_copy(y2_h.at[pl.ds(0, c)], rows, sem).wait()

    def process(i, slot):
      rows, colb, _ = slots[slot]
      base = i * c
      for j in range(c // L):
        colb[pl.ds(j * L, L)] = colT[pl.ds(base + j * L, L)]
      wait_gather(slot)

      @pl.when(i + 1 < nchunk)
      def _():
        start_gather(i + 1, 1 - slot)

      for j in range(c // L):
        _scale_rows(rows, normT[pl.ds(base + j * L, L)], j, nslice)
      pltpu.sync_copy(rows, acc_sh.at[colb], add=True)

    start_gather(0, 0)

    def seg_body(i2, _):
      for par in range(2):
        i = 2 * i2 + par

        @pl.when(i < nchunk)
        def _():
          process(i, par)

      return 0

    lax.fori_loop(0, (nchunk + 1) // 2, seg_body, 0)
    plsc.subcore_barrier()

    @pl.when(sid < ntiles_io)
    def _():
      pltpu.sync_copy(acc_sh.at[pl.ds(r0, rslice)],
                      s2_h.at[pl.ds(cid * n + r0, rslice)])

  return k(row, col, norm, y2)


def _tc_matmul(x, wt, bn):
  n, kdim = x.shape
  m = wt.shape[1]

  def body(x_ref, w_ref, o_ref):
    o_ref[...] = jnp.dot(x_ref[...], w_ref[...],
                         preferred_element_type=F32)

  return pl.pallas_call(
      body,
      grid=(n // bn,),
      in_specs=[
          pl.BlockSpec((bn, kdim), lambda i: (i, 0)),
          pl.BlockSpec((kdim, m), lambda i: (0, 0)),
      ],
      out_specs=pl.BlockSpec((bn, m), lambda i: (i, 0)),
      out_shape=jax.ShapeDtypeStruct((n, m), F32),
  )(x, wt)


def _tc_mid(t1, s1, b1, w2t, bn):
  n = t1.shape[0]
  hid = s1.shape[1]
  m = w2t.shape[1]

  def body(t1_ref, s1a_ref, s1b_ref, b1_ref, w_ref, o_ref):
    h = t1_ref[:, hid:] + s1a_ref[...] + s1b_ref[...] + b1_ref[...]
    h = jnp.maximum(h, 0.0)
    o_ref[...] = jnp.dot(h, w_ref[...], preferred_element_type=F32)

  return pl.pallas_call(
      body,
      grid=(n // bn,),
      in_specs=[
          pl.BlockSpec((bn, 2 * hid), lambda i: (i, 0)),
          pl.BlockSpec((bn, hid), lambda i: (i, 0)),
          pl.BlockSpec((bn, hid), lambda i, n_blk=n // bn: (i + n_blk, 0)),
          pl.BlockSpec((1, hid), lambda i: (0, 0)),
          pl.BlockSpec((hid, m), lambda i: (0, 0)),
      ],
      out_specs=pl.BlockSpec((bn, m), lambda i: (i, 0)),
      out_shape=jax.ShapeDtypeStruct((n, m), F32),
  )(t1, s1, s1, b1.reshape(1, hid), w2t)


def _tc_final(hw0, s2, b2, bn):
  n, ncls = hw0.shape
  d2 = s2.shape[1]

  def body(h_ref, s2a_ref, s2b_ref, b2_ref, o_ref):
    o_ref[...] = (h_ref[...] + s2a_ref[:, :ncls] + s2b_ref[:, :ncls]
                  + b2_ref[...])

  return pl.pallas_call(
      body,
      grid=(n // bn,),
      in_specs=[
          pl.BlockSpec((bn, ncls), lambda i: (i, 0)),
          pl.BlockSpec((bn, d2), lambda i: (i, 0)),
          pl.BlockSpec((bn, d2), lambda i, n_blk=n // bn: (i + n_blk, 0)),
          pl.BlockSpec((1, ncls), lambda i: (0, 0)),
      ],
      out_specs=pl.BlockSpec((bn, ncls), lambda i: (i, 0)),
      out_shape=jax.ShapeDtypeStruct((n, ncls), F32),
  )(hw0, s2, s2, b2.reshape(1, ncls))


@jax.jit
def kernel(x, edge_index, edge_weight, W0_1, W1_1, b1, W0_2, W1_2, b2):
  n, _ = x.shape
  e = edge_index.shape[1]
  hid = W0_1.shape[0]
  ncls = W0_2.shape[0]
  d2 = 48
  c = 80
  bn = 1000

  row = edge_index[0]
  col = edge_index[1]

  wc = jnp.concatenate([W1_1, W0_1], axis=0).T
  t1 = _tc_matmul(x, wc, bn)
  y1 = t1[:, :hid]

  norm, s1 = _sc_layer1(n, e, hid, c, row, col, edge_weight, y1)

  w2c = jnp.zeros((hid, d2 + ncls), F32)
  w2c = w2c.at[:, :ncls].set(W1_2.T)
  w2c = w2c.at[:, d2:].set(W0_2.T)
  t2 = _tc_mid(t1, s1, b1, w2c, bn)
  y2 = t2[:, :d2]
  hw0 = t2[:, d2:]

  s2 = _sc_layer2(n, e, d2, c, row, col, norm, y2)

  return _tc_final(hw0, s2, b2, bn)


# speedup vs baseline: 21.9760x; 1.0327x over previous
import functools

import jax
import jax.numpy as jnp
from jax import lax
from jax.experimental import pallas as pl
from jax.experimental.pallas import tpu as pltpu
from jax.experimental.pallas import tpu_sc as plsc

NC = 2
NS = 16
L = 16

F32 = jnp.float32
I32 = jnp.int32


def _rsqrt_newton(x):
  bits = lax.bitcast_convert_type(x, I32)
  seed = lax.bitcast_convert_type(jnp.int32(0x5F3759DF) - (bits >> 1), F32)
  xh = x * 0.5
  y = seed
  for _ in range(3):
    y = y * (1.5 - xh * y * y)
  return y


def _zero_fill(ref, nwords):
  z = jnp.zeros((L,), F32)

  def body(i, _):
    ref[pl.ds(i * L, L)] = z
    return 0

  lax.fori_loop(0, nwords // L, body, 0)


def _zero_fill2(ref, nrows, ncols):
  z = jnp.zeros((L,), F32)
  nslice = ncols // L

  def body(i, _):
    for k in range(nslice):
      ref[i, pl.ds(k * L, L)] = z
    return 0

  lax.fori_loop(0, nrows, body, 0)


def _lane_bcast(v, lane):
  return lax.squeeze(lax.slice(v, (lane,), (lane + 1,)), (0,))


def _scale_rows(rows_ref, norm16, j, nslice):
  for lane in range(L):
    e = j * L + lane
    s = _lane_bcast(norm16, lane)
    for k in range(nslice):
      sl = pl.ds(k * L, L)
      rows_ref[e, sl] = rows_ref[e, sl] * s


def _sc_layer1(n, e, d, c, row, col, w, y1):
  ept = e // (NC * NS)
  nchunk = ept // c
  rslice = 1000
  ntiles_io = n // rslice
  nslice = d // L

  mesh = plsc.VectorSubcoreMesh(core_axis_name="c", subcore_axis_name="s")

  @functools.partial(
      pl.kernel,
      out_type=(
          jax.ShapeDtypeStruct((e,), F32),
          jax.ShapeDtypeStruct((2 * n, d), F32),
      ),
      mesh=mesh,
      compiler_params=pltpu.CompilerParams(needs_layout_passes=False, use_tc_tiling_on_sc=False),
      scratch_types=dict(
          deg_sh=pltpu.VMEM_SHARED((n,), F32),
          acc_sh=pltpu.VMEM_SHARED((n, d), F32),
          dinv_v=pltpu.VMEM((n,), F32),
          rowT=pltpu.VMEM((ept,), I32),
          colT=pltpu.VMEM((ept,), I32),
          wT=pltpu.VMEM((ept,), F32),
          rowTo=pltpu.VMEM((ept // 5,), I32),
          colTo=pltpu.VMEM((ept // 5,), I32),
          wTo=pltpu.VMEM((ept // 5,), F32),
          normT=pltpu.VMEM((ept,), F32),
          rows_a=pltpu.VMEM((c, d), F32),
          rows_b=pltpu.VMEM((c, d), F32),
          rowb_a=pltpu.VMEM((c,), I32),
          rowb_b=pltpu.VMEM((c,), I32),
          colb_a=pltpu.VMEM((c,), I32),
          colb_b=pltpu.VMEM((c,), I32),
          wb_a=pltpu.VMEM((c,), F32),
          wb_b=pltpu.VMEM((c,), F32),
          gsem_a=pltpu.SemaphoreType.DMA,
          gsem_b=pltpu.SemaphoreType.DMA,
          ssem_a=pltpu.SemaphoreType.DMA,
          ssem_b=pltpu.SemaphoreType.DMA,
      ),
  )
  def k(row_h, col_h, w_h, y1_h, norm_h, s1_h, *, deg_sh, acc_sh, dinv_v,
        rowT, colT, wT, rowTo, colTo, wTo, normT, rows_a, rows_b,
        rowb_a, rowb_b, colb_a, colb_b, wb_a, wb_b, gsem_a, gsem_b,
        ssem_a, ssem_b):
    cid = lax.axis_index("c")
    sid = lax.axis_index("s")
    gid = cid * NS + sid
    obid = (1 - cid) * NS + sid

    pltpu.sync_copy(row_h.at[pl.ds(gid * ept, ept)], rowT)
    pltpu.sync_copy(col_h.at[pl.ds(gid * ept, ept)], colT)
    pltpu.sync_copy(w_h.at[pl.ds(gid * ept, ept)], wT)

    _zero_fill(dinv_v, n)
    _zero_fill2(rows_a, c, d)

    r0 = sid * rslice
    nfull = rslice // c
    rem = rslice - nfull * c

    @pl.when(sid < ntiles_io)
    def _():
      pltpu.sync_copy(dinv_v.at[pl.ds(0, rslice)],
                      deg_sh.at[pl.ds(r0, rslice)])
      for b in range(nfull):
        pltpu.sync_copy(rows_a, acc_sh.at[pl.ds(r0 + b * c, c)])
      if rem:
        pltpu.sync_copy(rows_a.at[pl.ds(0, rem)],
                        acc_sh.at[pl.ds(r0 + nfull * c, rem)])

    plsc.subcore_barrier()

    dslots = ((rowb_a, wb_a, ssem_a), (rowb_b, wb_b, ssem_b))

    def deg_wait(slot):
      rb, wbf, sem = dslots[slot]
      pltpu.make_async_copy(wbf, deg_sh.at[rb], sem).wait()

    def deg_chunks(rT_, cT_, wT_, count):
      def deg_body(i2, _):
        for par in range(2):
          i = 2 * i2 + par

          @pl.when(i < count)
          def _():
            rb, wbf, sem = dslots[par]

            @pl.when(i >= 2)
            def _():
              deg_wait(par)

            base = i * c
            for j in range(c // L):
              srcsl = pl.ds(base + j * L, L)
              dst = pl.ds(j * L, L)
              rv, cv, wv = rT_[srcsl], cT_[srcsl], wT_[srcsl]
              rb[dst] = rv
              wbf[dst] = jnp.where(rv == cv, 0.0, wv)
            pltpu.async_copy(wbf, deg_sh.at[rb], sem, add=True)

        return 0

      lax.fori_loop(0, (count + 1) // 2, deg_body, 0)
      deg_wait(0)
      deg_wait(1)

    deg_chunks(rowT, colT, wT, nchunk)
    piece = ept // 5
    for p in range(5):
      pltpu.sync_copy(row_h.at[pl.ds(obid * ept + p * piece, piece)], rowTo)
      pltpu.sync_copy(col_h.at[pl.ds(obid * ept + p * piece, piece)], colTo)
      pltpu.sync_copy(w_h.at[pl.ds(obid * ept + p * piece, piece)], wTo)
      deg_chunks(rowTo, colTo, wTo, piece // c)
    plsc.subcore_barrier()

    pltpu.sync_copy(deg_sh, dinv_v)

    def dinv_body(i, _):
      sl = pl.ds(i * L, L)
      dv = dinv_v[sl]
      dinv_v[sl] = jnp.where(dv > 0.0, _rsqrt_newton(jnp.maximum(dv, 1e-30)),
                             0.0)
      return 0

    lax.fori_loop(0, n // L, dinv_body, 0)

    slots = ((rows_a, colb_a, gsem_a, ssem_a), (rows_b, colb_b, gsem_b,
                                                ssem_b))

    def start_gather(i, slot):
      rows, _, sem, _ = slots[slot]
      pltpu.async_copy(y1_h.at[rowT.at[pl.ds(i * c, c)]], rows, sem)

    def wait_gather(slot):
      rows, _, sem, _ = slots[slot]
      pltpu.make_async_copy(y1_h.at[pl.ds(0, c)], rows, sem).wait()

    def wait_scatter(slot):
      rows, colb, _, sem = slots[slot]
      pltpu.make_async_copy(rows, acc_sh.at[colb], sem).wait()

    def process(i, slot):
      rows, colb, _, ssem = slots[slot]
      base = i * c
      for j in range(c // L):
        src = pl.ds(base + j * L, L)
        rv, cv, wv = rowT[src], colT[src], wT[src]
        dr = plsc.load_gather(dinv_v, [rv])
        dc = plsc.load_gather(dinv_v, [cv])
        weff = jnp.where(rv == cv, 0.0, wv)
        normT[src] = -(dr * weff * dc)
        colb[pl.ds(j * L, L)] = cv
      wait_gather(slot)

      @pl.when(i + 1 < nchunk)
      def _():
        @pl.when(i >= 1)
        def _():
          wait_scatter(1 - slot)

        start_gather(i + 1, 1 - slot)

      for j in range(c // L):
        _scale_rows(rows, normT[pl.ds(base + j * L, L)], j, nslice)
      pltpu.async_copy(rows, acc_sh.at[colb], ssem, add=True)

    start_gather(0, 0)

    def seg_body(i2, _):
      for par in range(2):
        i = 2 * i2 + par

        @pl.when(i < nchunk)
        def _():
          process(i, par)

      return 0

    lax.fori_loop(0, (nchunk + 1) // 2, seg_body, 0)
    wait_scatter(0)
    wait_scatter(1)
    pltpu.sync_copy(normT, norm_h.at[pl.ds(gid * ept, ept)])
    plsc.subcore_barrier()

    @pl.when(sid < ntiles_io)
    def _():
      pltpu.sync_copy(acc_sh.at[pl.ds(r0, rslice)],
                      s1_h.at[pl.ds(cid * n + r0, rslice)])

  return k(row, col, w, y1)


def _sc_layer2(n, e, d, c, row, col, norm, y2):
  ept = e // (NC * NS)
  nchunk = ept // c
  rslice = 1000
  ntiles_io = n // rslice
  nslice = d // L

  mesh = plsc.VectorSubcoreMesh(core_axis_name="c", subcore_axis_name="s")

  @functools.partial(
      pl.kernel,
      out_type=jax.ShapeDtypeStruct((2 * n, d), F32),
      mesh=mesh,
      compiler_params=pltpu.CompilerParams(needs_layout_passes=False, use_tc_tiling_on_sc=False),
      scratch_types=dict(
          acc_sh=pltpu.VMEM_SHARED((n, d), F32),
          rowT=pltpu.VMEM((ept,), I32),
          colT=pltpu.VMEM((ept,), I32),
          normT=pltpu.VMEM((ept,), F32),
          rows_a=pltpu.VMEM((c, d), F32),
          rows_b=pltpu.VMEM((c, d), F32),
          colb_a=pltpu.VMEM((c,), I32),
          colb_b=pltpu.VMEM((c,), I32),
          gsem_a=pltpu.SemaphoreType.DMA,
          gsem_b=pltpu.SemaphoreType.DMA,
          ssem_a=pltpu.SemaphoreType.DMA,
          ssem_b=pltpu.SemaphoreType.DMA,
      ),
  )
  def k(row_h, col_h, norm_h, y2_h, s2_h, *, acc_sh, rowT, colT, normT,
        rows_a, rows_b, colb_a, colb_b, gsem_a, gsem_b, ssem_a, ssem_b):
    cid = lax.axis_index("c")
    sid = lax.axis_index("s")
    gid = cid * NS + sid

    pltpu.sync_copy(row_h.at[pl.ds(gid * ept, ept)], rowT)
    pltpu.sync_copy(col_h.at[pl.ds(gid * ept, ept)], colT)
    pltpu.sync_copy(norm_h.at[pl.ds(gid * ept, ept)], normT)

    _zero_fill2(rows_a, c, d)
    r0 = sid * rslice
    nfull = rslice // c
    rem = rslice - nfull * c

    @pl.when(sid < ntiles_io)
    def _():
      for b in range(nfull):
        pltpu.sync_copy(rows_a, acc_sh.at[pl.ds(r0 + b * c, c)])
      if rem:
        pltpu.sync_copy(rows_a.at[pl.ds(0, rem)],
                        acc_sh.at[pl.ds(r0 + nfull * c, rem)])

    plsc.subcore_barrier()

    slots = ((rows_a, colb_a, gsem_a, ssem_a), (rows_b, colb_b, gsem_b,
                                                ssem_b))

    def start_gather(i, slot):
      rows, _, sem, _ = slots[slot]
      pltpu.async_copy(y2_h.at[rowT.at[pl.ds(i * c, c)]], rows, sem)

    def wait_gather(slot):
      rows, _, sem, _ = slots[slot]
      pltpu.make_async_copy(y2_h.at[pl.ds(0, c)], rows, sem).wait()

    def wait_scatter(slot):
      rows, colb, _, sem = slots[slot]
      pltpu.make_async_copy(rows, acc_sh.at[colb], sem).wait()

    def process(i, slot):
      rows, colb, _, ssem = slots[slot]
      base = i * c
      for j in range(c // L):
        colb[pl.ds(j * L, L)] = colT[pl.ds(base + j * L, L)]
      wait_gather(slot)

      @pl.when(i + 1 < nchunk)
      def _():
        @pl.when(i >= 1)
        def _():
          wait_scatter(1 - slot)

        start_gather(i + 1, 1 - slot)

      for j in range(c // L):
        _scale_rows(rows, normT[pl.ds(base + j * L, L)], j, nslice)
      pltpu.async_copy(rows, acc_sh.at[colb], ssem, add=True)

    start_gather(0, 0)

    def seg_body(i2, _):
      for par in range(2):
        i = 2 * i2 + par

        @pl.when(i < nchunk)
        def _():
          process(i, par)

      return 0

    lax.fori_loop(0, (nchunk + 1) // 2, seg_body, 0)
    wait_scatter(0)
    wait_scatter(1)
    plsc.subcore_barrier()

    @pl.when(sid < ntiles_io)
    def _():
      pltpu.sync_copy(acc_sh.at[pl.ds(r0, rslice)],
                      s2_h.at[pl.ds(cid * n + r0, rslice)])

  return k(row, col, norm, y2)


def _tc_matmul(x, wt, bn):
  n, kdim = x.shape
  m = wt.shape[1]

  def body(x_ref, w_ref, o_ref):
    o_ref[...] = jnp.dot(x_ref[...], w_ref[...],
                         preferred_element_type=F32)

  return pl.pallas_call(
      body,
      grid=(n // bn,),
      in_specs=[
          pl.BlockSpec((bn, kdim), lambda i: (i, 0)),
          pl.BlockSpec((kdim, m), lambda i: (0, 0)),
      ],
      out_specs=pl.BlockSpec((bn, m), lambda i: (i, 0)),
      out_shape=jax.ShapeDtypeStruct((n, m), F32),
  )(x, wt)


def _tc_mid(t1, s1, b1, w2t, bn):
  n = t1.shape[0]
  hid = s1.shape[1]
  m = w2t.shape[1]

  def body(t1_ref, s1a_ref, s1b_ref, b1_ref, w_ref, o_ref):
    h = t1_ref[:, hid:] + s1a_ref[...] + s1b_ref[...] + b1_ref[...]
    h = jnp.maximum(h, 0.0)
    o_ref[...] = jnp.dot(h, w_ref[...], preferred_element_type=F32)

  return pl.pallas_call(
      body,
      grid=(n // bn,),
      in_specs=[
          pl.BlockSpec((bn, 2 * hid), lambda i: (i, 0)),
          pl.BlockSpec((bn, hid), lambda i: (i, 0)),
          pl.BlockSpec((bn, hid), lambda i, n_blk=n // bn: (i + n_blk, 0)),
          pl.BlockSpec((1, hid), lambda i: (0, 0)),
          pl.BlockSpec((hid, m), lambda i: (0, 0)),
      ],
      out_specs=pl.BlockSpec((bn, m), lambda i: (i, 0)),
      out_shape=jax.ShapeDtypeStruct((n, m), F32),
  )(t1, s1, s1, b1.reshape(1, hid), w2t)


def _tc_final(hw0, s2, b2, bn):
  n, ncls = hw0.shape
  d2 = s2.shape[1]

  def body(h_ref, s2a_ref, s2b_ref, b2_ref, o_ref):
    o_ref[...] = (h_ref[...] + s2a_ref[:, :ncls] + s2b_ref[:, :ncls]
                  + b2_ref[...])

  return pl.pallas_call(
      body,
      grid=(n // bn,),
      in_specs=[
          pl.BlockSpec((bn, ncls), lambda i: (i, 0)),
          pl.BlockSpec((bn, d2), lambda i: (i, 0)),
          pl.BlockSpec((bn, d2), lambda i, n_blk=n // bn: (i + n_blk, 0)),
          pl.BlockSpec((1, ncls), lambda i: (0, 0)),
      ],
      out_specs=pl.BlockSpec((bn, ncls), lambda i: (i, 0)),
      out_shape=jax.ShapeDtypeStruct((n, ncls), F32),
  )(hw0, s2, s2, b2.reshape(1, ncls))


@jax.jit
def kernel(x, edge_index, edge_weight, W0_1, W1_1, b1, W0_2, W1_2, b2):
  n, _ = x.shape
  e = edge_index.shape[1]
  hid = W0_1.shape[0]
  ncls = W0_2.shape[0]
  d2 = 48
  c = 80
  bn = 1000

  row = edge_index[0]
  col = edge_index[1]

  wc = jnp.concatenate([W1_1, W0_1], axis=0).T
  t1 = _tc_matmul(x, wc, bn)
  y1 = t1[:, :hid]

  norm, s1 = _sc_layer1(n, e, hid, c, row, col, edge_weight, y1)

  w2c = jnp.zeros((hid, d2 + ncls), F32)
  w2c = w2c.at[:, :ncls].set(W1_2.T)
  w2c = w2c.at[:, d2:].set(W0_2.T)
  t2 = _tc_mid(t1, s1, b1, w2c, bn)
  y2 = t2[:, :d2]
  hw0 = t2[:, d2:]

  s2 = _sc_layer2(n, e, d2, c, row, col, norm, y2)

  return _tc_final(hw0, s2, b2, bn)


# speedup vs baseline: 25.5050x; 1.1606x over previous
import functools

import jax
import jax.numpy as jnp
from jax import lax
from jax.experimental import pallas as pl
from jax.experimental.pallas import tpu as pltpu
from jax.experimental.pallas import tpu_sc as plsc

NC = 2
NS = 16
L = 16

F32 = jnp.float32
I32 = jnp.int32


def _rsqrt_newton(x):
  bits = lax.bitcast_convert_type(x, I32)
  seed = lax.bitcast_convert_type(jnp.int32(0x5F3759DF) - (bits >> 1), F32)
  xh = x * 0.5
  y = seed
  for _ in range(3):
    y = y * (1.5 - xh * y * y)
  return y


def _zero_fill(ref, nwords):
  z = jnp.zeros((L,), F32)

  def body(i, _):
    ref[pl.ds(i * L, L)] = z
    return 0

  lax.fori_loop(0, nwords // L, body, 0)


def _zero_fill2(ref, nrows, ncols):
  z = jnp.zeros((L,), F32)
  nslice = ncols // L

  def body(i, _):
    for k in range(nslice):
      ref[i, pl.ds(k * L, L)] = z
    return 0

  lax.fori_loop(0, nrows, body, 0)


def _lane_bcast(v, lane):
  return lax.squeeze(lax.slice(v, (lane,), (lane + 1,)), (0,))


def _scale_rows(rows_ref, norm16, j, nslice):
  for lane in range(L):
    e = j * L + lane
    s = _lane_bcast(norm16, lane)
    for k in range(nslice):
      sl = pl.ds(k * L, L)
      rows_ref[e, sl] = rows_ref[e, sl] * s


def _sc_layer1(n, e, d, c, row, col, w, y1):
  ept = e // (NC * NS)
  nchunk = ept // c
  rslice = 1000
  ntiles_io = n // rslice
  nslice = d // L

  mesh = plsc.VectorSubcoreMesh(core_axis_name="c", subcore_axis_name="s")

  @functools.partial(
      pl.kernel,
      out_type=(
          jax.ShapeDtypeStruct((e,), F32),
          jax.ShapeDtypeStruct((2 * n, d), F32),
      ),
      mesh=mesh,
      compiler_params=pltpu.CompilerParams(needs_layout_passes=False, use_tc_tiling_on_sc=False),
      scratch_types=dict(
          deg_sh=pltpu.VMEM_SHARED((n,), F32),
          acc_sh=pltpu.VMEM_SHARED((n, d), F32),
          dinv_v=pltpu.VMEM((n,), F32),
          rowT=pltpu.VMEM((ept,), I32),
          colT=pltpu.VMEM((ept,), I32),
          wT=pltpu.VMEM((ept,), F32),
          rowTo=pltpu.VMEM((ept // 5,), I32),
          colTo=pltpu.VMEM((ept // 5,), I32),
          wTo=pltpu.VMEM((ept // 5,), F32),
          normT=pltpu.VMEM((ept,), F32),
          rows_a=pltpu.VMEM((c, d), F32),
          rows_b=pltpu.VMEM((c, d), F32),
          rowb_a=pltpu.VMEM((c,), I32),
          rowb_b=pltpu.VMEM((c,), I32),
          colb_a=pltpu.VMEM((c,), I32),
          colb_b=pltpu.VMEM((c,), I32),
          wb_a=pltpu.VMEM((c,), F32),
          wb_b=pltpu.VMEM((c,), F32),
          gsem_a=pltpu.SemaphoreType.DMA,
          gsem_b=pltpu.SemaphoreType.DMA,
          ssem_a=pltpu.SemaphoreType.DMA,
          ssem_b=pltpu.SemaphoreType.DMA,
      ),
  )
  def k(row_h, col_h, w_h, y1_h, norm_h, s1_h, *, deg_sh, acc_sh, dinv_v,
        rowT, colT, wT, rowTo, colTo, wTo, normT, rows_a, rows_b,
        rowb_a, rowb_b, colb_a, colb_b, wb_a, wb_b, gsem_a, gsem_b,
        ssem_a, ssem_b):
    cid = lax.axis_index("c")
    sid = lax.axis_index("s")
    gid = cid * NS + sid
    obid = (1 - cid) * NS + sid

    pltpu.sync_copy(row_h.at[pl.ds(gid * ept, ept)], rowT)
    pltpu.sync_copy(col_h.at[pl.ds(gid * ept, ept)], colT)
    pltpu.sync_copy(w_h.at[pl.ds(gid * ept, ept)], wT)

    _zero_fill(dinv_v, n)
    _zero_fill2(rows_a, c, d)

    r0 = sid * rslice
    nfull = rslice // c
    rem = rslice - nfull * c

    @pl.when(sid < ntiles_io)
    def _():
      pltpu.sync_copy(dinv_v.at[pl.ds(0, rslice)],
                      deg_sh.at[pl.ds(r0, rslice)])
      for b in range(nfull):
        pltpu.sync_copy(rows_a, acc_sh.at[pl.ds(r0 + b * c, c)])
      if rem:
        pltpu.sync_copy(rows_a.at[pl.ds(0, rem)],
                        acc_sh.at[pl.ds(r0 + nfull * c, rem)])

    plsc.subcore_barrier()

    dslots = ((rowb_a, wb_a, ssem_a), (rowb_b, wb_b, ssem_b))

    def deg_wait(slot):
      rb, wbf, sem = dslots[slot]
      pltpu.make_async_copy(wbf, deg_sh.at[rb], sem).wait()

    def deg_chunks(rT_, cT_, wT_, count):
      def deg_body(i2, _):
        for par in range(2):
          i = 2 * i2 + par

          @pl.when(i < count)
          def _():
            rb, wbf, sem = dslots[par]

            @pl.when(i >= 2)
            def _():
              deg_wait(par)

            base = i * c
            for j in range(c // L):
              srcsl = pl.ds(base + j * L, L)
              dst = pl.ds(j * L, L)
              rv, cv, wv = rT_[srcsl], cT_[srcsl], wT_[srcsl]
              rb[dst] = rv
              wbf[dst] = jnp.where(rv == cv, 0.0, wv)
            pltpu.async_copy(wbf, deg_sh.at[rb], sem, add=True)

        return 0

      lax.fori_loop(0, (count + 1) // 2, deg_body, 0)
      deg_wait(0)
      deg_wait(1)

    deg_chunks(rowT, colT, wT, nchunk)
    piece = ept // 5
    for p in range(5):
      pltpu.sync_copy(row_h.at[pl.ds(obid * ept + p * piece, piece)], rowTo)
      pltpu.sync_copy(col_h.at[pl.ds(obid * ept + p * piece, piece)], colTo)
      pltpu.sync_copy(w_h.at[pl.ds(obid * ept + p * piece, piece)], wTo)
      deg_chunks(rowTo, colTo, wTo, piece // c)
    plsc.subcore_barrier()

    pltpu.sync_copy(deg_sh, dinv_v)

    def dinv_body(i, _):
      sl = pl.ds(i * L, L)
      dv = dinv_v[sl]
      dinv_v[sl] = jnp.where(dv > 0.0, _rsqrt_newton(jnp.maximum(dv, 1e-30)),
                             0.0)
      return 0

    lax.fori_loop(0, n // L, dinv_body, 0)

    slots = ((rows_a, colb_a, gsem_a, ssem_a), (rows_b, colb_b, gsem_b,
                                                ssem_b))

    def start_gather(i, slot):
      rows, _, sem, _ = slots[slot]
      pltpu.async_copy(y1_h.at[rowT.at[pl.ds(i * c, c)]], rows, sem)

    def wait_gather(slot):
      rows, _, sem, _ = slots[slot]
      pltpu.make_async_copy(y1_h.at[pl.ds(0, c)], rows, sem).wait()

    def wait_scatter(slot):
      rows, colb, _, sem = slots[slot]
      pltpu.make_async_copy(rows, acc_sh.at[colb], sem).wait()

    def process(i, slot):
      rows, colb, _, ssem = slots[slot]
      base = i * c
      for j in range(c // L):
        src = pl.ds(base + j * L, L)
        rv, cv, wv = rowT[src], colT[src], wT[src]
        dr = plsc.load_gather(dinv_v, [rv])
        dc = plsc.load_gather(dinv_v, [cv])
        weff = jnp.where(rv == cv, 0.0, wv)
        normT[src] = -(dr * weff * dc)
        colb[pl.ds(j * L, L)] = cv
      wait_gather(slot)

      @pl.when(i + 1 < nchunk)
      def _():
        @pl.when(i >= 1)
        def _():
          wait_scatter(1 - slot)

        start_gather(i + 1, 1 - slot)

      for j in range(c // L):
        _scale_rows(rows, normT[pl.ds(base + j * L, L)], j, nslice)
      pltpu.async_copy(rows, acc_sh.at[colb], ssem, add=True)

    start_gather(0, 0)

    def seg_body(i2, _):
      for par in range(2):
        i = 2 * i2 + par

        @pl.when(i < nchunk)
        def _():
          process(i, par)

      return 0

    lax.fori_loop(0, (nchunk + 1) // 2, seg_body, 0)
    wait_scatter(0)
    wait_scatter(1)
    pltpu.sync_copy(normT, norm_h.at[pl.ds(gid * ept, ept)])
    plsc.subcore_barrier()

    @pl.when(sid < ntiles_io)
    def _():
      pltpu.sync_copy(acc_sh.at[pl.ds(r0, rslice)],
                      s1_h.at[pl.ds(cid * n + r0, rslice)])

  return k(row, col, w, y1)


def _sc_layer2(n, e, d, c, row, col, norm, y2):
  ept = e // (NC * NS)
  nchunk = ept // c
  rslice = 1000
  ntiles_io = n // rslice
  nslice = d // L

  mesh = plsc.VectorSubcoreMesh(core_axis_name="c", subcore_axis_name="s")

  @functools.partial(
      pl.kernel,
      out_type=jax.ShapeDtypeStruct((2 * n, d), F32),
      mesh=mesh,
      compiler_params=pltpu.CompilerParams(needs_layout_passes=False, use_tc_tiling_on_sc=False),
      scratch_types=dict(
          acc_sh=pltpu.VMEM_SHARED((n, d), F32),
          y2_sh=pltpu.VMEM_SHARED((n, d), F32),
          rowT=pltpu.VMEM((ept,), I32),
          colT=pltpu.VMEM((ept,), I32),
          normT=pltpu.VMEM((ept,), F32),
          rows_a=pltpu.VMEM((c, d), F32),
          rows_b=pltpu.VMEM((c, d), F32),
          colb_a=pltpu.VMEM((c,), I32),
          colb_b=pltpu.VMEM((c,), I32),
          gsem_a=pltpu.SemaphoreType.DMA,
          gsem_b=pltpu.SemaphoreType.DMA,
          ssem_a=pltpu.SemaphoreType.DMA,
          ssem_b=pltpu.SemaphoreType.DMA,
      ),
  )
  def k(row_h, col_h, norm_h, y2_h, s2_h, *, acc_sh, y2_sh, rowT, colT,
        normT, rows_a, rows_b, colb_a, colb_b, gsem_a, gsem_b, ssem_a,
        ssem_b):
    cid = lax.axis_index("c")
    sid = lax.axis_index("s")
    gid = cid * NS + sid

    pltpu.sync_copy(row_h.at[pl.ds(gid * ept, ept)], rowT)
    pltpu.sync_copy(col_h.at[pl.ds(gid * ept, ept)], colT)
    pltpu.sync_copy(norm_h.at[pl.ds(gid * ept, ept)], normT)

    _zero_fill2(rows_a, c, d)
    r1 = sid * rslice

    @pl.when(sid < ntiles_io)
    def _():
      pltpu.sync_copy(y2_h.at[pl.ds(r1, rslice)],
                      y2_sh.at[pl.ds(r1, rslice)])

    r0 = sid * rslice
    nfull = rslice // c
    rem = rslice - nfull * c

    @pl.when(sid < ntiles_io)
    def _():
      for b in range(nfull):
        pltpu.sync_copy(rows_a, acc_sh.at[pl.ds(r0 + b * c, c)])
      if rem:
        pltpu.sync_copy(rows_a.at[pl.ds(0, rem)],
                        acc_sh.at[pl.ds(r0 + nfull * c, rem)])

    plsc.subcore_barrier()

    slots = ((rows_a, colb_a, gsem_a, ssem_a), (rows_b, colb_b, gsem_b,
                                                ssem_b))

    def start_gather(i, slot):
      rows, _, sem, _ = slots[slot]
      pltpu.async_copy(y2_sh.at[rowT.at[pl.ds(i * c, c)]], rows, sem)

    def wait_gather(slot):
      rows, _, sem, _ = slots[slot]
      pltpu.make_async_copy(y2_h.at[pl.ds(0, c)], rows, sem).wait()

    def wait_scatter(slot):
      rows, colb, _, sem = slots[slot]
      pltpu.make_async_copy(rows, acc_sh.at[colb], sem).wait()

    def process(i, slot):
      rows, colb, _, ssem = slots[slot]
      base = i * c
      for j in range(c // L):
        colb[pl.ds(j * L, L)] = colT[pl.ds(base + j * L, L)]
      wait_gather(slot)

      @pl.when(i + 1 < nchunk)
      def _():
        @pl.when(i >= 1)
        def _():
          wait_scatter(1 - slot)

        start_gather(i + 1, 1 - slot)

      for j in range(c // L):
        _scale_rows(rows, normT[pl.ds(base + j * L, L)], j, nslice)
      pltpu.async_copy(rows, acc_sh.at[colb], ssem, add=True)

    start_gather(0, 0)

    def seg_body(i2, _):
      for par in range(2):
        i = 2 * i2 + par

        @pl.when(i < nchunk)
        def _():
          process(i, par)

      return 0

    lax.fori_loop(0, (nchunk + 1) // 2, seg_body, 0)
    wait_scatter(0)
    wait_scatter(1)
    plsc.subcore_barrier()

    @pl.when(sid < ntiles_io)
    def _():
      pltpu.sync_copy(acc_sh.at[pl.ds(r0, rslice)],
                      s2_h.at[pl.ds(cid * n + r0, rslice)])

  return k(row, col, norm, y2)


def _tc_matmul2(x, wt, m1, bn):
  n, kdim = x.shape
  m = wt.shape[1]

  def body(x_ref, w_ref, o1_ref, o2_ref):
    res = jnp.dot(x_ref[...], w_ref[...], preferred_element_type=F32)
    o1_ref[...] = res[:, :m1]
    o2_ref[...] = res[:, m1:]

  return pl.pallas_call(
      body,
      grid=(n // bn,),
      in_specs=[
          pl.BlockSpec((bn, kdim), lambda i: (i, 0)),
          pl.BlockSpec((kdim, m), lambda i: (0, 0)),
      ],
      out_specs=[
          pl.BlockSpec((bn, m1), lambda i: (i, 0)),
          pl.BlockSpec((bn, m - m1), lambda i: (i, 0)),
      ],
      out_shape=[
          jax.ShapeDtypeStruct((n, m1), F32),
          jax.ShapeDtypeStruct((n, m - m1), F32),
      ],
  )(x, wt)


def _tc_mid(xw0, s1, b1, w2t, m1, bn):
  n = xw0.shape[0]
  hid = s1.shape[1]
  m = w2t.shape[1]

  def body(x_ref, s1a_ref, s1b_ref, b1_ref, w_ref, o1_ref, o2_ref):
    h = x_ref[...] + s1a_ref[...] + s1b_ref[...] + b1_ref[...]
    h = jnp.maximum(h, 0.0)
    res = jnp.dot(h, w_ref[...], preferred_element_type=F32)
    o1_ref[...] = res[:, :m1]
    o2_ref[...] = res[:, m1:]

  return pl.pallas_call(
      body,
      grid=(n // bn,),
      in_specs=[
          pl.BlockSpec((bn, hid), lambda i: (i, 0)),
          pl.BlockSpec((bn, hid), lambda i: (i, 0)),
          pl.BlockSpec((bn, hid), lambda i, n_blk=n // bn: (i + n_blk, 0)),
          pl.BlockSpec((1, hid), lambda i: (0, 0)),
          pl.BlockSpec((hid, m), lambda i: (0, 0)),
      ],
      out_specs=[
          pl.BlockSpec((bn, m1), lambda i: (i, 0)),
          pl.BlockSpec((bn, m - m1), lambda i: (i, 0)),
      ],
      out_shape=[
          jax.ShapeDtypeStruct((n, m1), F32),
          jax.ShapeDtypeStruct((n, m - m1), F32),
      ],
  )(xw0, s1, s1, b1.reshape(1, hid), w2t)


def _tc_final(hw0, s2, b2, bn):
  n, ncls = hw0.shape
  d2 = s2.shape[1]

  def body(h_ref, s2a_ref, s2b_ref, b2_ref, o_ref):
    o_ref[...] = (h_ref[...] + s2a_ref[:, :ncls] + s2b_ref[:, :ncls]
                  + b2_ref[...])

  return pl.pallas_call(
      body,
      grid=(n // bn,),
      in_specs=[
          pl.BlockSpec((bn, ncls), lambda i: (i, 0)),
          pl.BlockSpec((bn, d2), lambda i: (i, 0)),
          pl.BlockSpec((bn, d2), lambda i, n_blk=n // bn: (i + n_blk, 0)),
          pl.BlockSpec((1, ncls), lambda i: (0, 0)),
      ],
      out_specs=pl.BlockSpec((bn, ncls), lambda i: (i, 0)),
      out_shape=jax.ShapeDtypeStruct((n, ncls), F32),
  )(hw0, s2, s2, b2.reshape(1, ncls))


@jax.jit
def kernel(x, edge_index, edge_weight, W0_1, W1_1, b1, W0_2, W1_2, b2):
  n, _ = x.shape
  e = edge_index.shape[1]
  hid = W0_1.shape[0]
  ncls = W0_2.shape[0]
  d2 = 48
  c = 80
  bn = 1000

  row = edge_index[0]
  col = edge_index[1]

  wc = jnp.concatenate([W1_1, W0_1], axis=0).T
  y1, xw0 = _tc_matmul2(x, wc, hid, bn)

  norm, s1 = _sc_layer1(n, e, hid, c, row, col, edge_weight, y1)

  w2c = jnp.zeros((hid, d2 + ncls), F32)
  w2c = w2c.at[:, :ncls].set(W1_2.T)
  w2c = w2c.at[:, d2:].set(W0_2.T)
  y2, hw0 = _tc_mid(xw0, s1, b1, w2c, d2, bn)

  s2 = _sc_layer2(n, e, d2, c, row, col, norm, y2)

  return _tc_final(hw0, s2, b2, bn)


# speedup vs baseline: 26.3250x; 1.0322x over previous
import functools

import jax
import jax.numpy as jnp
from jax import lax
from jax.experimental import pallas as pl
from jax.experimental.pallas import tpu as pltpu
from jax.experimental.pallas import tpu_sc as plsc

NC = 2
NS = 16
L = 16

F32 = jnp.float32
I32 = jnp.int32


def _rsqrt_newton(x):
  bits = lax.bitcast_convert_type(x, I32)
  seed = lax.bitcast_convert_type(jnp.int32(0x5F3759DF) - (bits >> 1), F32)
  xh = x * 0.5
  y = seed
  for _ in range(3):
    y = y * (1.5 - xh * y * y)
  return y


def _zero_fill(ref, nwords):
  z = jnp.zeros((L,), F32)

  def body(i, _):
    ref[pl.ds(i * L, L)] = z
    return 0

  lax.fori_loop(0, nwords // L, body, 0)


def _zero_fill2(ref, nrows, ncols):
  z = jnp.zeros((L,), F32)
  nslice = ncols // L

  def body(i, _):
    for k in range(nslice):
      ref[i, pl.ds(k * L, L)] = z
    return 0

  lax.fori_loop(0, nrows, body, 0)


def _lane_bcast(v, lane):
  return lax.squeeze(lax.slice(v, (lane,), (lane + 1,)), (0,))


def _scale_rows(rows_ref, norm16, j, nslice):
  for lane in range(L):
    e = j * L + lane
    s = _lane_bcast(norm16, lane)
    for k in range(nslice):
      sl = pl.ds(k * L, L)
      rows_ref[e, sl] = rows_ref[e, sl] * s


def _sc_layer1(n, e, d, c, row, col, w, y1):
  ept = e // (NC * NS)
  nchunk = ept // c
  half0 = (nchunk // 2 + 1) * c
  half1 = ept - half0
  rslice = 1000
  ntiles_io = n // rslice
  nslice = d // L

  mesh = plsc.VectorSubcoreMesh(core_axis_name="c", subcore_axis_name="s")

  @functools.partial(
      pl.kernel,
      out_type=(
          jax.ShapeDtypeStruct((e,), F32),
          jax.ShapeDtypeStruct((2 * n, d), F32),
      ),
      mesh=mesh,
      compiler_params=pltpu.CompilerParams(needs_layout_passes=False, use_tc_tiling_on_sc=False),
      scratch_types=dict(
          deg_sh=pltpu.VMEM_SHARED((n,), F32),
          acc_sh=pltpu.VMEM_SHARED((n, d), F32),
          y1_sh=pltpu.VMEM_SHARED((n, d), F32),
          dinv_v=pltpu.VMEM((n,), F32),
          rowT=pltpu.VMEM((half0,), I32),
          colT=pltpu.VMEM((half0,), I32),
          wT=pltpu.VMEM((half0,), F32),
          rowTo=pltpu.VMEM((ept // 5,), I32),
          colTo=pltpu.VMEM((ept // 5,), I32),
          wTo=pltpu.VMEM((ept // 5,), F32),
          normT=pltpu.VMEM((half0,), F32),
          rows_a=pltpu.VMEM((c, d), F32),
          rows_b=pltpu.VMEM((c, d), F32),
          rowb_a=pltpu.VMEM((c,), I32),
          rowb_b=pltpu.VMEM((c,), I32),
          colb_a=pltpu.VMEM((c,), I32),
          colb_b=pltpu.VMEM((c,), I32),
          wb_a=pltpu.VMEM((c,), F32),
          wb_b=pltpu.VMEM((c,), F32),
          gsem_a=pltpu.SemaphoreType.DMA,
          gsem_b=pltpu.SemaphoreType.DMA,
          ssem_a=pltpu.SemaphoreType.DMA,
          ssem_b=pltpu.SemaphoreType.DMA,
      ),
  )
  def k(row_h, col_h, w_h, y1_h, norm_h, s1_h, *, deg_sh, acc_sh, y1_sh,
        dinv_v, rowT, colT, wT, rowTo, colTo, wTo, normT, rows_a, rows_b,
        rowb_a, rowb_b, colb_a, colb_b, wb_a, wb_b, gsem_a, gsem_b,
        ssem_a, ssem_b):
    cid = lax.axis_index("c")
    sid = lax.axis_index("s")
    gid = cid * NS + sid
    obid = (1 - cid) * NS + sid

    def load_own(off, cnt):
      pltpu.sync_copy(row_h.at[pl.ds(gid * ept + off, cnt)],
                      rowT.at[pl.ds(0, cnt)])
      pltpu.sync_copy(col_h.at[pl.ds(gid * ept + off, cnt)],
                      colT.at[pl.ds(0, cnt)])
      pltpu.sync_copy(w_h.at[pl.ds(gid * ept + off, cnt)],
                      wT.at[pl.ds(0, cnt)])

    load_own(0, half0)

    _zero_fill(dinv_v, n)
    _zero_fill2(rows_a, c, d)

    r0 = sid * rslice
    nfull = rslice // c
    rem = rslice - nfull * c

    @pl.when(sid < ntiles_io)
    def _():
      pltpu.sync_copy(dinv_v.at[pl.ds(0, rslice)],
                      deg_sh.at[pl.ds(r0, rslice)])
      pltpu.sync_copy(y1_h.at[pl.ds(r0, rslice)],
                      y1_sh.at[pl.ds(r0, rslice)])
      for b in range(nfull):
        pltpu.sync_copy(rows_a, acc_sh.at[pl.ds(r0 + b * c, c)])
      if rem:
        pltpu.sync_copy(rows_a.at[pl.ds(0, rem)],
                        acc_sh.at[pl.ds(r0 + nfull * c, rem)])

    plsc.subcore_barrier()

    dslots = ((rowb_a, wb_a, ssem_a), (rowb_b, wb_b, ssem_b))

    def deg_wait(slot):
      rb, wbf, sem = dslots[slot]
      pltpu.make_async_copy(wbf, deg_sh.at[rb], sem).wait()

    def deg_chunks(rT_, cT_, wT_, count):
      def deg_body(i2, _):
        for par in range(2):
          i = 2 * i2 + par

          @pl.when(i < count)
          def _():
            rb, wbf, sem = dslots[par]

            @pl.when(i >= 2)
            def _():
              deg_wait(par)

            base = i * c
            for j in range(c // L):
              srcsl = pl.ds(base + j * L, L)
              dst = pl.ds(j * L, L)
              rv, cv, wv = rT_[srcsl], cT_[srcsl], wT_[srcsl]
              rb[dst] = rv
              wbf[dst] = jnp.where(rv == cv, 0.0, wv)
            pltpu.async_copy(wbf, deg_sh.at[rb], sem, add=True)

        return 0

      lax.fori_loop(0, (count + 1) // 2, deg_body, 0)
      deg_wait(0)
      deg_wait(1)

    deg_chunks(rowT, colT, wT, half0 // c)
    load_own(half0, half1)
    deg_chunks(rowT, colT, wT, half1 // c)
    piece = ept // 5
    for p in range(5):
      pltpu.sync_copy(row_h.at[pl.ds(obid * ept + p * piece, piece)], rowTo)
      pltpu.sync_copy(col_h.at[pl.ds(obid * ept + p * piece, piece)], colTo)
      pltpu.sync_copy(w_h.at[pl.ds(obid * ept + p * piece, piece)], wTo)
      deg_chunks(rowTo, colTo, wTo, piece // c)
    plsc.subcore_barrier()

    pltpu.sync_copy(deg_sh, dinv_v)

    def dinv_body(i, _):
      sl = pl.ds(i * L, L)
      dv = dinv_v[sl]
      dinv_v[sl] = jnp.where(dv > 0.0, _rsqrt_newton(jnp.maximum(dv, 1e-30)),
                             0.0)
      return 0

    lax.fori_loop(0, n // L, dinv_body, 0)

    slots = ((rows_a, colb_a, gsem_a, ssem_a), (rows_b, colb_b, gsem_b,
                                                ssem_b))

    def start_gather(i, slot):
      rows, _, sem, _ = slots[slot]
      pltpu.async_copy(y1_sh.at[rowT.at[pl.ds(i * c, c)]], rows, sem)

    def wait_gather(slot):
      rows, _, sem, _ = slots[slot]
      pltpu.make_async_copy(y1_h.at[pl.ds(0, c)], rows, sem).wait()

    def wait_scatter(slot):
      rows, colb, _, sem = slots[slot]
      pltpu.make_async_copy(rows, acc_sh.at[colb], sem).wait()

    def run_half(off, cnt):
      hchunk = cnt // c

      def process(i, slot):
        rows, colb, _, ssem = slots[slot]
        base = i * c
        for j in range(c // L):
          srcsl = pl.ds(base + j * L, L)
          rv, cv, wv = rowT[srcsl], colT[srcsl], wT[srcsl]
          dr = plsc.load_gather(dinv_v, [rv])
          dc = plsc.load_gather(dinv_v, [cv])
          weff = jnp.where(rv == cv, 0.0, wv)
          normT[srcsl] = -(dr * weff * dc)
          colb[pl.ds(j * L, L)] = cv
        wait_gather(slot)

        @pl.when(i + 1 < hchunk)
        def _():
          @pl.when(i >= 1)
          def _():
            wait_scatter(1 - slot)

          start_gather(i + 1, 1 - slot)

        for j in range(c // L):
          _scale_rows(rows, normT[pl.ds(base + j * L, L)], j, nslice)
        pltpu.async_copy(rows, acc_sh.at[colb], ssem, add=True)

      start_gather(0, 0)

      def seg_body(i2, _):
        for par in range(2):
          i = 2 * i2 + par

          @pl.when(i < hchunk)
          def _():
            process(i, par)

        return 0

      lax.fori_loop(0, (hchunk + 1) // 2, seg_body, 0)
      wait_scatter(0)
      wait_scatter(1)
      pltpu.sync_copy(normT.at[pl.ds(0, cnt)],
                      norm_h.at[pl.ds(gid * ept + off, cnt)])

    load_own(0, half0)
    run_half(0, half0)
    load_own(half0, half1)
    run_half(half0, half1)
    plsc.subcore_barrier()

    @pl.when(sid < ntiles_io)
    def _():
      pltpu.sync_copy(acc_sh.at[pl.ds(r0, rslice)],
                      s1_h.at[pl.ds(cid * n + r0, rslice)])

  return k(row, col, w, y1)


def _sc_layer2(n, e, d, c, row, col, norm, y2):
  ept = e // (NC * NS)
  nchunk = ept // c
  rslice = 1000
  ntiles_io = n // rslice
  nslice = d // L

  mesh = plsc.VectorSubcoreMesh(core_axis_name="c", subcore_axis_name="s")

  @functools.partial(
      pl.kernel,
      out_type=jax.ShapeDtypeStruct((2 * n, d), F32),
      mesh=mesh,
      compiler_params=pltpu.CompilerParams(needs_layout_passes=False, use_tc_tiling_on_sc=False),
      scratch_types=dict(
          acc_sh=pltpu.VMEM_SHARED((n, d), F32),
          y2_sh=pltpu.VMEM_SHARED((n, d), F32),
          rowT=pltpu.VMEM((ept,), I32),
          colT=pltpu.VMEM((ept,), I32),
          normT=pltpu.VMEM((ept,), F32),
          rows_a=pltpu.VMEM((c, d), F32),
          rows_b=pltpu.VMEM((c, d), F32),
          colb_a=pltpu.VMEM((c,), I32),
          colb_b=pltpu.VMEM((c,), I32),
          gsem_a=pltpu.SemaphoreType.DMA,
          gsem_b=pltpu.SemaphoreType.DMA,
          ssem_a=pltpu.SemaphoreType.DMA,
          ssem_b=pltpu.SemaphoreType.DMA,
      ),
  )
  def k(row_h, col_h, norm_h, y2_h, s2_h, *, acc_sh, y2_sh, rowT, colT,
        normT, rows_a, rows_b, colb_a, colb_b, gsem_a, gsem_b, ssem_a,
        ssem_b):
    cid = lax.axis_index("c")
    sid = lax.axis_index("s")
    gid = cid * NS + sid

    pltpu.sync_copy(row_h.at[pl.ds(gid * ept, ept)], rowT)
    pltpu.sync_copy(col_h.at[pl.ds(gid * ept, ept)], colT)
    pltpu.sync_copy(norm_h.at[pl.ds(gid * ept, ept)], normT)

    _zero_fill2(rows_a, c, d)
    r1 = sid * rslice

    @pl.when(sid < ntiles_io)
    def _():
      pltpu.sync_copy(y2_h.at[pl.ds(r1, rslice)],
                      y2_sh.at[pl.ds(r1, rslice)])

    r0 = sid * rslice
    nfull = rslice // c
    rem = rslice - nfull * c

    @pl.when(sid < ntiles_io)
    def _():
      for b in range(nfull):
        pltpu.sync_copy(rows_a, acc_sh.at[pl.ds(r0 + b * c, c)])
      if rem:
        pltpu.sync_copy(rows_a.at[pl.ds(0, rem)],
                        acc_sh.at[pl.ds(r0 + nfull * c, rem)])

    plsc.subcore_barrier()

    slots = ((rows_a, colb_a, gsem_a, ssem_a), (rows_b, colb_b, gsem_b,
                                                ssem_b))

    def start_gather(i, slot):
      rows, _, sem, _ = slots[slot]
      pltpu.async_copy(y2_sh.at[rowT.at[pl.ds(i * c, c)]], rows, sem)

    def wait_gather(slot):
      rows, _, sem, _ = slots[slot]
      pltpu.make_async_copy(y2_h.at[pl.ds(0, c)], rows, sem).wait()

    def wait_scatter(slot):
      rows, colb, _, sem = slots[slot]
      pltpu.make_async_copy(rows, acc_sh.at[colb], sem).wait()

    def process(i, slot):
      rows, colb, _, ssem = slots[slot]
      base = i * c
      for j in range(c // L):
        colb[pl.ds(j * L, L)] = colT[pl.ds(base + j * L, L)]
      wait_gather(slot)

      @pl.when(i + 1 < nchunk)
      def _():
        @pl.when(i >= 1)
        def _():
          wait_scatter(1 - slot)

        start_gather(i + 1, 1 - slot)

      for j in range(c // L):
        _scale_rows(rows, normT[pl.ds(base + j * L, L)], j, nslice)
      pltpu.async_copy(rows, acc_sh.at[colb], ssem, add=True)

    start_gather(0, 0)

    def seg_body(i2, _):
      for par in range(2):
        i = 2 * i2 + par

        @pl.when(i < nchunk)
        def _():
          process(i, par)

      return 0

    lax.fori_loop(0, (nchunk + 1) // 2, seg_body, 0)
    wait_scatter(0)
    wait_scatter(1)
    plsc.subcore_barrier()

    @pl.when(sid < ntiles_io)
    def _():
      pltpu.sync_copy(acc_sh.at[pl.ds(r0, rslice)],
                      s2_h.at[pl.ds(cid * n + r0, rslice)])

  return k(row, col, norm, y2)


def _tc_matmul2(x, wt, m1, bn):
  n, kdim = x.shape
  m = wt.shape[1]

  def body(x_ref, w_ref, o1_ref, o2_ref):
    res = jnp.dot(x_ref[...], w_ref[...], preferred_element_type=F32)
    o1_ref[...] = res[:, :m1]
    o2_ref[...] = res[:, m1:]

  return pl.pallas_call(
      body,
      grid=(n // bn,),
      in_specs=[
          pl.BlockSpec((bn, kdim), lambda i: (i, 0)),
          pl.BlockSpec((kdim, m), lambda i: (0, 0)),
      ],
      out_specs=[
          pl.BlockSpec((bn, m1), lambda i: (i, 0)),
          pl.BlockSpec((bn, m - m1), lambda i: (i, 0)),
      ],
      out_shape=[
          jax.ShapeDtypeStruct((n, m1), F32),
          jax.ShapeDtypeStruct((n, m - m1), F32),
      ],
  )(x, wt)


def _tc_mid(xw0, s1, b1, w2t, m1, bn):
  n = xw0.shape[0]
  hid = s1.shape[1]
  m = w2t.shape[1]

  def body(x_ref, s1a_ref, s1b_ref, b1_ref, w_ref, o1_ref, o2_ref):
    h = x_ref[...] + s1a_ref[...] + s1b_ref[...] + b1_ref[...]
    h = jnp.maximum(h, 0.0)
    res = jnp.dot(h, w_ref[...], preferred_element_type=F32)
    o1_ref[...] = res[:, :m1]
    o2_ref[...] = res[:, m1:]

  return pl.pallas_call(
      body,
      grid=(n // bn,),
      in_specs=[
          pl.BlockSpec((bn, hid), lambda i: (i, 0)),
          pl.BlockSpec((bn, hid), lambda i: (i, 0)),
          pl.BlockSpec((bn, hid), lambda i, n_blk=n // bn: (i + n_blk, 0)),
          pl.BlockSpec((1, hid), lambda i: (0, 0)),
          pl.BlockSpec((hid, m), lambda i: (0, 0)),
      ],
      out_specs=[
          pl.BlockSpec((bn, m1), lambda i: (i, 0)),
          pl.BlockSpec((bn, m - m1), lambda i: (i, 0)),
      ],
      out_shape=[
          jax.ShapeDtypeStruct((n, m1), F32),
          jax.ShapeDtypeStruct((n, m - m1), F32),
      ],
  )(xw0, s1, s1, b1.reshape(1, hid), w2t)


def _tc_final(hw0, s2, b2, bn):
  n, ncls = hw0.shape
  d2 = s2.shape[1]

  def body(h_ref, s2a_ref, s2b_ref, b2_ref, o_ref):
    o_ref[...] = (h_ref[...] + s2a_ref[:, :ncls] + s2b_ref[:, :ncls]
                  + b2_ref[...])

  return pl.pallas_call(
      body,
      grid=(n // bn,),
      in_specs=[
          pl.BlockSpec((bn, ncls), lambda i: (i, 0)),
          pl.BlockSpec((bn, d2), lambda i: (i, 0)),
          pl.BlockSpec((bn, d2), lambda i, n_blk=n // bn: (i + n_blk, 0)),
          pl.BlockSpec((1, ncls), lambda i: (0, 0)),
      ],
      out_specs=pl.BlockSpec((bn, ncls), lambda i: (i, 0)),
      out_shape=jax.ShapeDtypeStruct((n, ncls), F32),
  )(hw0, s2, s2, b2.reshape(1, ncls))


@jax.jit
def kernel(x, edge_index, edge_weight, W0_1, W1_1, b1, W0_2, W1_2, b2):
  n, _ = x.shape
  e = edge_index.shape[1]
  hid = W0_1.shape[0]
  ncls = W0_2.shape[0]
  d2 = 48
  c = 80
  bn = 1000

  row = edge_index[0]
  col = edge_index[1]

  wc = jnp.concatenate([W1_1, W0_1], axis=0).T
  y1, xw0 = _tc_matmul2(x, wc, hid, bn)

  norm, s1 = _sc_layer1(n, e, hid, c, row, col, edge_weight, y1)

  w2c = jnp.zeros((hid, d2 + ncls), F32)
  w2c = w2c.at[:, :ncls].set(W1_2.T)
  w2c = w2c.at[:, d2:].set(W0_2.T)
  y2, hw0 = _tc_mid(xw0, s1, b1, w2c, d2, bn)

  s2 = _sc_layer2(n, e, d2, c, row, col, norm, y2)

  return _tc_final(hw0, s2, b2, bn)


# speedup vs baseline: 26.8337x; 1.0193x over previous
import functools

import jax
import jax.numpy as jnp
from jax import lax
from jax.experimental import pallas as pl
from jax.experimental.pallas import tpu as pltpu
from jax.experimental.pallas import tpu_sc as plsc

NC = 2
NS = 16
L = 16

F32 = jnp.float32
I32 = jnp.int32


def _rsqrt_newton(x):
  bits = lax.bitcast_convert_type(x, I32)
  seed = lax.bitcast_convert_type(jnp.int32(0x5F3759DF) - (bits >> 1), F32)
  xh = x * 0.5
  y = seed
  for _ in range(3):
    y = y * (1.5 - xh * y * y)
  return y


def _zero_fill(ref, nwords):
  z = jnp.zeros((L,), F32)

  def body(i, _):
    ref[pl.ds(i * L, L)] = z
    return 0

  lax.fori_loop(0, nwords // L, body, 0)


def _zero_fill2(ref, nrows, ncols):
  z = jnp.zeros((L,), F32)
  nslice = ncols // L

  def body(i, _):
    for k in range(nslice):
      ref[i, pl.ds(k * L, L)] = z
    return 0

  lax.fori_loop(0, nrows, body, 0)


def _lane_bcast(v, lane):
  return lax.squeeze(lax.slice(v, (lane,), (lane + 1,)), (0,))


def _scale_rows(rows_ref, norm16, j, nslice):
  for lane in range(L):
    e = j * L + lane
    s = _lane_bcast(norm16, lane)
    for k in range(nslice):
      sl = pl.ds(k * L, L)
      rows_ref[e, sl] = rows_ref[e, sl] * s


def _sc_layer1(n, e, d, c, row, col, w, y1):
  ept = e // (NC * NS)
  nchunk = ept // c
  half0 = (nchunk // 2 + 1) * c
  half1 = ept - half0
  rslice = 1000
  ntiles_io = n // rslice
  nslice = d // L

  mesh = plsc.VectorSubcoreMesh(core_axis_name="c", subcore_axis_name="s")

  @functools.partial(
      pl.kernel,
      out_type=(
          jax.ShapeDtypeStruct((e,), F32),
          jax.ShapeDtypeStruct((2 * n, d), F32),
      ),
      mesh=mesh,
      compiler_params=pltpu.CompilerParams(needs_layout_passes=False, use_tc_tiling_on_sc=False),
      scratch_types=dict(
          deg_sh=pltpu.VMEM_SHARED((n,), F32),
          acc_sh=pltpu.VMEM_SHARED((n, d), F32),
          y1_sh=pltpu.VMEM_SHARED((n, d), F32),
          dinv_v=pltpu.VMEM((n,), F32),
          rowT=pltpu.VMEM((half0,), I32),
          colT=pltpu.VMEM((half0,), I32),
          wT=pltpu.VMEM((half0,), F32),
          rowTo=pltpu.VMEM((ept // 5,), I32),
          colTo=pltpu.VMEM((ept // 5,), I32),
          wTo=pltpu.VMEM((ept // 5,), F32),
          normT=pltpu.VMEM((half0,), F32),
          rows_a=pltpu.VMEM((c, d), F32),
          rows_b=pltpu.VMEM((c, d), F32),
          rowb_a=pltpu.VMEM((c,), I32),
          rowb_b=pltpu.VMEM((c,), I32),
          colb_a=pltpu.VMEM((c,), I32),
          colb_b=pltpu.VMEM((c,), I32),
          wb_a=pltpu.VMEM((c,), F32),
          wb_b=pltpu.VMEM((c,), F32),
          gsem_a=pltpu.SemaphoreType.DMA,
          gsem_b=pltpu.SemaphoreType.DMA,
          ssem_a=pltpu.SemaphoreType.DMA,
          ssem_b=pltpu.SemaphoreType.DMA,
      ),
  )
  def k(row_h, col_h, w_h, y1_h, norm_h, s1_h, *, deg_sh, acc_sh, y1_sh,
        dinv_v, rowT, colT, wT, rowTo, colTo, wTo, normT, rows_a, rows_b,
        rowb_a, rowb_b, colb_a, colb_b, wb_a, wb_b, gsem_a, gsem_b,
        ssem_a, ssem_b):
    cid = lax.axis_index("c")
    sid = lax.axis_index("s")
    gid = cid * NS + sid
    obid = (1 - cid) * NS + sid

    def load_own(off, cnt):
      pltpu.sync_copy(row_h.at[pl.ds(gid * ept + off, cnt)],
                      rowT.at[pl.ds(0, cnt)])
      pltpu.sync_copy(col_h.at[pl.ds(gid * ept + off, cnt)],
                      colT.at[pl.ds(0, cnt)])
      pltpu.sync_copy(w_h.at[pl.ds(gid * ept + off, cnt)],
                      wT.at[pl.ds(0, cnt)])

    load_own(0, half0)

    _zero_fill(dinv_v, n)
    _zero_fill2(rows_a, c, d)

    r0 = sid * rslice
    nfull = rslice // c
    rem = rslice - nfull * c

    @pl.when(sid < ntiles_io)
    def _():
      pltpu.sync_copy(dinv_v.at[pl.ds(0, rslice)],
                      deg_sh.at[pl.ds(r0, rslice)])
      pltpu.async_copy(y1_h.at[pl.ds(r0, rslice)],
                       y1_sh.at[pl.ds(r0, rslice)], gsem_a)
      for b in range(nfull):
        pltpu.sync_copy(rows_a, acc_sh.at[pl.ds(r0 + b * c, c)])
      if rem:
        pltpu.sync_copy(rows_a.at[pl.ds(0, rem)],
                        acc_sh.at[pl.ds(r0 + nfull * c, rem)])

    plsc.subcore_barrier()

    dslots = ((rowb_a, wb_a, ssem_a), (rowb_b, wb_b, ssem_b))

    def deg_wait(slot):
      rb, wbf, sem = dslots[slot]
      pltpu.make_async_copy(wbf, deg_sh.at[rb], sem).wait()

    def deg_chunks(rT_, cT_, wT_, count):
      def deg_body(i2, _):
        for par in range(2):
          i = 2 * i2 + par

          @pl.when(i < count)
          def _():
            rb, wbf, sem = dslots[par]

            @pl.when(i >= 2)
            def _():
              deg_wait(par)

            base = i * c
            for j in range(c // L):
              srcsl = pl.ds(base + j * L, L)
              dst = pl.ds(j * L, L)
              rv, cv, wv = rT_[srcsl], cT_[srcsl], wT_[srcsl]
              rb[dst] = rv
              wbf[dst] = jnp.where(rv == cv, 0.0, wv)
            pltpu.async_copy(wbf, deg_sh.at[rb], sem, add=True)

        return 0

      lax.fori_loop(0, (count + 1) // 2, deg_body, 0)
      deg_wait(0)
      deg_wait(1)

    deg_chunks(rowT, colT, wT, half0 // c)
    load_own(half0, half1)
    deg_chunks(rowT, colT, wT, half1 // c)
    piece = ept // 5
    for p in range(5):
      pltpu.sync_copy(row_h.at[pl.ds(obid * ept + p * piece, piece)], rowTo)
      pltpu.sync_copy(col_h.at[pl.ds(obid * ept + p * piece, piece)], colTo)
      pltpu.sync_copy(w_h.at[pl.ds(obid * ept + p * piece, piece)], wTo)
      deg_chunks(rowTo, colTo, wTo, piece // c)

    @pl.when(sid < ntiles_io)
    def _():
      pltpu.make_async_copy(y1_h.at[pl.ds(r0, rslice)],
                            y1_sh.at[pl.ds(r0, rslice)], gsem_a).wait()

    plsc.subcore_barrier()

    pltpu.sync_copy(deg_sh, dinv_v)

    def dinv_body(i, _):
      sl = pl.ds(i * L, L)
      dv = dinv_v[sl]
      dinv_v[sl] = jnp.where(dv > 0.0, _rsqrt_newton(jnp.maximum(dv, 1e-30)),
                             0.0)
      return 0

    lax.fori_loop(0, n // L, dinv_body, 0)

    slots = ((rows_a, colb_a, gsem_a, ssem_a), (rows_b, colb_b, gsem_b,
                                                ssem_b))

    def start_gather(i, slot):
      rows, _, sem, _ = slots[slot]
      pltpu.async_copy(y1_sh.at[rowT.at[pl.ds(i * c, c)]], rows, sem)

    def wait_gather(slot):
      rows, _, sem, _ = slots[slot]
      pltpu.make_async_copy(y1_h.at[pl.ds(0, c)], rows, sem).wait()

    def wait_scatter(slot):
      rows, colb, _, sem = slots[slot]
      pltpu.make_async_copy(rows, acc_sh.at[colb], sem).wait()

    def run_half(off, cnt):
      hchunk = cnt // c

      def process(i, slot):
        rows, colb, _, ssem = slots[slot]
        base = i * c
        for j in range(c // L):
          srcsl = pl.ds(base + j * L, L)
          rv, cv, wv = rowT[srcsl], colT[srcsl], wT[srcsl]
          dr = plsc.load_gather(dinv_v, [rv])
          dc = plsc.load_gather(dinv_v, [cv])
          weff = jnp.where(rv == cv, 0.0, wv)
          normT[srcsl] = -(dr * weff * dc)
          colb[pl.ds(j * L, L)] = cv
        wait_gather(slot)

        @pl.when(i + 1 < hchunk)
        def _():
          @pl.when(i >= 1)
          def _():
            wait_scatter(1 - slot)

          start_gather(i + 1, 1 - slot)

        for j in range(c // L):
          _scale_rows(rows, normT[pl.ds(base + j * L, L)], j, nslice)
        pltpu.async_copy(rows, acc_sh.at[colb], ssem, add=True)

      start_gather(0, 0)

      def seg_body(i2, _):
        for par in range(2):
          i = 2 * i2 + par

          @pl.when(i < hchunk)
          def _():
            process(i, par)

        return 0

      lax.fori_loop(0, (hchunk + 1) // 2, seg_body, 0)
      wait_scatter(0)
      wait_scatter(1)
      pltpu.sync_copy(normT.at[pl.ds(0, cnt)],
                      norm_h.at[pl.ds(gid * ept + off, cnt)])

    load_own(0, half0)
    run_half(0, half0)
    load_own(half0, half1)
    run_half(half0, half1)
    plsc.subcore_barrier()

    @pl.when(sid < ntiles_io)
    def _():
      pltpu.sync_copy(acc_sh.at[pl.ds(r0, rslice)],
                      s1_h.at[pl.ds(cid * n + r0, rslice)])

  return k(row, col, w, y1)


def _sc_layer2(n, e, d, c, row, col, norm, y2):
  ept = e // (NC * NS)
  nchunk = ept // c
  rslice = 1000
  ntiles_io = n // rslice
  nslice = d // L

  mesh = plsc.VectorSubcoreMesh(core_axis_name="c", subcore_axis_name="s")

  @functools.partial(
      pl.kernel,
      out_type=jax.ShapeDtypeStruct((2 * n, d), F32),
      mesh=mesh,
      compiler_params=pltpu.CompilerParams(needs_layout_passes=False, use_tc_tiling_on_sc=False),
      scratch_types=dict(
          acc_sh=pltpu.VMEM_SHARED((n, d), F32),
          y2_sh=pltpu.VMEM_SHARED((n, d), F32),
          rowT=pltpu.VMEM((ept,), I32),
          colT=pltpu.VMEM((ept,), I32),
          normT=pltpu.VMEM((ept,), F32),
          rows_a=pltpu.VMEM((c, d), F32),
          rows_b=pltpu.VMEM((c, d), F32),
          colb_a=pltpu.VMEM((c,), I32),
          colb_b=pltpu.VMEM((c,), I32),
          gsem_a=pltpu.SemaphoreType.DMA,
          gsem_b=pltpu.SemaphoreType.DMA,
          ssem_a=pltpu.SemaphoreType.DMA,
          ssem_b=pltpu.SemaphoreType.DMA,
      ),
  )
  def k(row_h, col_h, norm_h, y2_h, s2_h, *, acc_sh, y2_sh, rowT, colT,
        normT, rows_a, rows_b, colb_a, colb_b, gsem_a, gsem_b, ssem_a,
        ssem_b):
    cid = lax.axis_index("c")
    sid = lax.axis_index("s")
    gid = cid * NS + sid

    pltpu.sync_copy(row_h.at[pl.ds(gid * ept, ept)], rowT)
    pltpu.sync_copy(col_h.at[pl.ds(gid * ept, ept)], colT)
    pltpu.sync_copy(norm_h.at[pl.ds(gid * ept, ept)], normT)

    r1 = sid * rslice

    @pl.when(sid < ntiles_io)
    def _():
      pltpu.async_copy(y2_h.at[pl.ds(r1, rslice)],
                       y2_sh.at[pl.ds(r1, rslice)], gsem_a)

    _zero_fill2(rows_a, c, d)

    r0 = sid * rslice
    nfull = rslice // c
    rem = rslice - nfull * c

    @pl.when(sid < ntiles_io)
    def _():
      for b in range(nfull):
        pltpu.sync_copy(rows_a, acc_sh.at[pl.ds(r0 + b * c, c)])
      if rem:
        pltpu.sync_copy(rows_a.at[pl.ds(0, rem)],
                        acc_sh.at[pl.ds(r0 + nfull * c, rem)])
      pltpu.make_async_copy(y2_h.at[pl.ds(r1, rslice)],
                            y2_sh.at[pl.ds(r1, rslice)], gsem_a).wait()

    plsc.subcore_barrier()

    slots = ((rows_a, colb_a, gsem_a, ssem_a), (rows_b, colb_b, gsem_b,
                                                ssem_b))

    def start_gather(i, slot):
      rows, _, sem, _ = slots[slot]
      pltpu.async_copy(y2_sh.at[rowT.at[pl.ds(i * c, c)]], rows, sem)

    def wait_gather(slot):
      rows, _, sem, _ = slots[slot]
      pltpu.make_async_copy(y2_h.at[pl.ds(0, c)], rows, sem).wait()

    def wait_scatter(slot):
      rows, colb, _, sem = slots[slot]
      pltpu.make_async_copy(rows, acc_sh.at[colb], sem).wait()

    def process(i, slot):
      rows, colb, _, ssem = slots[slot]
      base = i * c
      for j in range(c // L):
        colb[pl.ds(j * L, L)] = colT[pl.ds(base + j * L, L)]
      wait_gather(slot)

      @pl.when(i + 1 < nchunk)
      def _():
        @pl.when(i >= 1)
        def _():
          wait_scatter(1 - slot)

        start_gather(i + 1, 1 - slot)

      for j in range(c // L):
        _scale_rows(rows, normT[pl.ds(base + j * L, L)], j, nslice)
      pltpu.async_copy(rows, acc_sh.at[colb], ssem, add=True)

    start_gather(0, 0)

    def seg_body(i2, _):
      for par in range(2):
        i = 2 * i2 + par

        @pl.when(i < nchunk)
        def _():
          process(i, par)

      return 0

    lax.fori_loop(0, (nchunk + 1) // 2, seg_body, 0)
    wait_scatter(0)
    wait_scatter(1)
    plsc.subcore_barrier()

    @pl.when(sid < ntiles_io)
    def _():
      pltpu.sync_copy(acc_sh.at[pl.ds(r0, rslice)],
                      s2_h.at[pl.ds(cid * n + r0, rslice)])

  return k(row, col, norm, y2)


def _dot_t(a, b):
  return lax.dot_general(a, b, (((1,), (1,)), ((), ())),
                         preferred_element_type=F32)


def _tc_matmul2(x, w1, w0, bn):
  n, kdim = x.shape
  m1 = w1.shape[0]
  m2 = w0.shape[0]

  def body(x_ref, w1_ref, w0_ref, o1_ref, o2_ref):
    xb = x_ref[...]
    o1_ref[...] = _dot_t(xb, w1_ref[...])
    o2_ref[...] = _dot_t(xb, w0_ref[...])

  return pl.pallas_call(
      body,
      grid=(n // bn,),
      in_specs=[
          pl.BlockSpec((bn, kdim), lambda i: (i, 0)),
          pl.BlockSpec((m1, kdim), lambda i: (0, 0)),
          pl.BlockSpec((m2, kdim), lambda i: (0, 0)),
      ],
      out_specs=[
          pl.BlockSpec((bn, m1), lambda i: (i, 0)),
          pl.BlockSpec((bn, m2), lambda i: (i, 0)),
      ],
      out_shape=[
          jax.ShapeDtypeStruct((n, m1), F32),
          jax.ShapeDtypeStruct((n, m2), F32),
      ],
  )(x, w1, w0)


def _tc_mid(xw0, s1, b1, w1_2, w0_2, d2, bn):
  n = xw0.shape[0]
  hid = s1.shape[1]
  ncls = w1_2.shape[0]

  def body(x_ref, s1a_ref, s1b_ref, b1_ref, w1_ref, w0_ref, o1_ref, o2_ref):
    h = x_ref[...] + s1a_ref[...] + s1b_ref[...] + b1_ref[...]
    h = jnp.maximum(h, 0.0)
    y2 = _dot_t(h, w1_ref[...])
    o1_ref[...] = jnp.concatenate(
        [y2, jnp.zeros((y2.shape[0], d2 - ncls), F32)], axis=1)
    o2_ref[...] = _dot_t(h, w0_ref[...])

  return pl.pallas_call(
      body,
      grid=(n // bn,),
      in_specs=[
          pl.BlockSpec((bn, hid), lambda i: (i, 0)),
          pl.BlockSpec((bn, hid), lambda i: (i, 0)),
          pl.BlockSpec((bn, hid), lambda i, n_blk=n // bn: (i + n_blk, 0)),
          pl.BlockSpec((1, hid), lambda i: (0, 0)),
          pl.BlockSpec((ncls, hid), lambda i: (0, 0)),
          pl.BlockSpec((ncls, hid), lambda i: (0, 0)),
      ],
      out_specs=[
          pl.BlockSpec((bn, d2), lambda i: (i, 0)),
          pl.BlockSpec((bn, ncls), lambda i: (i, 0)),
      ],
      out_shape=[
          jax.ShapeDtypeStruct((n, d2), F32),
          jax.ShapeDtypeStruct((n, ncls), F32),
      ],
  )(xw0, s1, s1, b1.reshape(1, hid), w1_2, w0_2)


def _tc_final(hw0, s2, b2, bn):
  n, ncls = hw0.shape
  d2 = s2.shape[1]

  def body(h_ref, s2a_ref, s2b_ref, b2_ref, o_ref):
    o_ref[...] = (h_ref[...] + s2a_ref[:, :ncls] + s2b_ref[:, :ncls]
                  + b2_ref[...])

  return pl.pallas_call(
      body,
      grid=(n // bn,),
      in_specs=[
          pl.BlockSpec((bn, ncls), lambda i: (i, 0)),
          pl.BlockSpec((bn, d2), lambda i: (i, 0)),
          pl.BlockSpec((bn, d2), lambda i, n_blk=n // bn: (i + n_blk, 0)),
          pl.BlockSpec((1, ncls), lambda i: (0, 0)),
      ],
      out_specs=pl.BlockSpec((bn, ncls), lambda i: (i, 0)),
      out_shape=jax.ShapeDtypeStruct((n, ncls), F32),
  )(hw0, s2, s2, b2.reshape(1, ncls))


@jax.jit
def kernel(x, edge_index, edge_weight, W0_1, W1_1, b1, W0_2, W1_2, b2):
  n, _ = x.shape
  e = edge_index.shape[1]
  hid = W0_1.shape[0]
  ncls = W0_2.shape[0]
  d2 = 48
  c = 80
  bn = 1000

  row = edge_index[0]
  col = edge_index[1]

  y1, xw0 = _tc_matmul2(x, W1_1, W0_1, bn)

  norm, s1 = _sc_layer1(n, e, hid, c, row, col, edge_weight, y1)

  y2, hw0 = _tc_mid(xw0, s1, b1, W1_2, W0_2, d2, bn)

  s2 = _sc_layer2(n, e, d2, c, row, col, norm, y2)

  return _tc_final(hw0, s2, b2, bn)


# speedup vs baseline: 27.0986x; 1.0099x over previous
import functools

import jax
import jax.numpy as jnp
from jax import lax
from jax.experimental import pallas as pl
from jax.experimental.pallas import tpu as pltpu
from jax.experimental.pallas import tpu_sc as plsc

NC = 2
NS = 16
L = 16

F32 = jnp.float32
I32 = jnp.int32


def _rsqrt_newton(x):
  bits = lax.bitcast_convert_type(x, I32)
  seed = lax.bitcast_convert_type(jnp.int32(0x5F3759DF) - (bits >> 1), F32)
  xh = x * 0.5
  y = seed
  for _ in range(3):
    y = y * (1.5 - xh * y * y)
  return y


def _zero_fill(ref, nwords):
  z = jnp.zeros((L,), F32)

  def body(i, _):
    ref[pl.ds(i * L, L)] = z
    return 0

  lax.fori_loop(0, nwords // L, body, 0)


def _zero_fill2(ref, nrows, ncols):
  z = jnp.zeros((L,), F32)
  nslice = ncols // L

  def body(i, _):
    for k in range(nslice):
      ref[i, pl.ds(k * L, L)] = z
    return 0

  lax.fori_loop(0, nrows, body, 0)


def _lane_bcast(v, lane):
  return lax.squeeze(lax.slice(v, (lane,), (lane + 1,)), (0,))


def _scale_rows(rows_ref, norm16, j, nslice):
  for lane in range(L):
    e = j * L + lane
    s = _lane_bcast(norm16, lane)
    for k in range(nslice):
      sl = pl.ds(k * L, L)
      rows_ref[e, sl] = rows_ref[e, sl] * s


def _sc_layer1(n, e, d, c, row, col, w, y1):
  ept = e // (NC * NS)
  nchunk = ept // c
  half0 = (nchunk // 2 + 1) * c
  half1 = ept - half0
  rslice = 1000
  ntiles_io = n // rslice
  nslice = d // L

  mesh = plsc.VectorSubcoreMesh(core_axis_name="c", subcore_axis_name="s")

  @functools.partial(
      pl.kernel,
      out_type=(
          jax.ShapeDtypeStruct((e,), F32),
          jax.ShapeDtypeStruct((2 * n, d), F32),
      ),
      mesh=mesh,
      compiler_params=pltpu.CompilerParams(needs_layout_passes=False, use_tc_tiling_on_sc=False),
      scratch_types=dict(
          deg_sh=pltpu.VMEM_SHARED((n,), F32),
          acc_sh=pltpu.VMEM_SHARED((n, d), F32),
          y1_sh=pltpu.VMEM_SHARED((n, d), F32),
          dinv_v=pltpu.VMEM((n,), F32),
          rowT=pltpu.VMEM((half0,), I32),
          colT=pltpu.VMEM((half0,), I32),
          wT=pltpu.VMEM((half0,), F32),
          rowTo=pltpu.VMEM((ept // 5,), I32),
          colTo=pltpu.VMEM((ept // 5,), I32),
          wTo=pltpu.VMEM((ept // 5,), F32),
          normT=pltpu.VMEM((half0,), F32),
          rows_a=pltpu.VMEM((c, d), F32),
          rows_b=pltpu.VMEM((c, d), F32),
          rowb_a=pltpu.VMEM((c,), I32),
          rowb_b=pltpu.VMEM((c,), I32),
          colb_a=pltpu.VMEM((c,), I32),
          colb_b=pltpu.VMEM((c,), I32),
          wb_a=pltpu.VMEM((c,), F32),
          wb_b=pltpu.VMEM((c,), F32),
          gsem_a=pltpu.SemaphoreType.DMA,
          gsem_b=pltpu.SemaphoreType.DMA,
          ssem_a=pltpu.SemaphoreType.DMA,
          ssem_b=pltpu.SemaphoreType.DMA,
      ),
  )
  def k(row_h, col_h, w_h, y1_h, norm_h, s1_h, *, deg_sh, acc_sh, y1_sh,
        dinv_v, rowT, colT, wT, rowTo, colTo, wTo, normT, rows_a, rows_b,
        rowb_a, rowb_b, colb_a, colb_b, wb_a, wb_b, gsem_a, gsem_b,
        ssem_a, ssem_b):
    cid = lax.axis_index("c")
    sid = lax.axis_index("s")
    gid = cid * NS + sid
    obid = (1 - cid) * NS + sid

    def load_own(off, cnt):
      pltpu.sync_copy(row_h.at[pl.ds(gid * ept + off, cnt)],
                      rowT.at[pl.ds(0, cnt)])
      pltpu.sync_copy(col_h.at[pl.ds(gid * ept + off, cnt)],
                      colT.at[pl.ds(0, cnt)])
      pltpu.sync_copy(w_h.at[pl.ds(gid * ept + off, cnt)],
                      wT.at[pl.ds(0, cnt)])

    load_own(0, half0)

    _zero_fill(dinv_v, n)
    _zero_fill2(rows_a, c, d)

    r0 = sid * rslice
    nfull = rslice // c
    rem = rslice - nfull * c

    @pl.when(sid < ntiles_io)
    def _():
      pltpu.sync_copy(dinv_v.at[pl.ds(0, rslice)],
                      deg_sh.at[pl.ds(r0, rslice)])
      pltpu.async_copy(y1_h.at[pl.ds(r0, rslice)],
                       y1_sh.at[pl.ds(r0, rslice)], gsem_a)
      for b in range(nfull):
        pltpu.sync_copy(rows_a, acc_sh.at[pl.ds(r0 + b * c, c)])
      if rem:
        pltpu.sync_copy(rows_a.at[pl.ds(0, rem)],
                        acc_sh.at[pl.ds(r0 + nfull * c, rem)])

    plsc.subcore_barrier()

    dslots = ((rowb_a, wb_a, ssem_a), (rowb_b, wb_b, ssem_b))

    def deg_wait(slot):
      rb, wbf, sem = dslots[slot]
      pltpu.make_async_copy(wbf, deg_sh.at[rb], sem).wait()

    def deg_chunks(rT_, cT_, wT_, count):
      def deg_body(i2, _):
        for par in range(2):
          i = 2 * i2 + par

          @pl.when(i < count)
          def _():
            rb, wbf, sem = dslots[par]

            @pl.when(i >= 2)
            def _():
              deg_wait(par)

            base = i * c
            for j in range(c // L):
              srcsl = pl.ds(base + j * L, L)
              dst = pl.ds(j * L, L)
              rv, cv, wv = rT_[srcsl], cT_[srcsl], wT_[srcsl]
              rb[dst] = rv
              wbf[dst] = jnp.where(rv == cv, 0.0, wv)
            pltpu.async_copy(wbf, deg_sh.at[rb], sem, add=True)

        return 0

      lax.fori_loop(0, (count + 1) // 2, deg_body, 0)
      deg_wait(0)
      deg_wait(1)

    deg_chunks(rowT, colT, wT, half0 // c)
    load_own(half0, half1)
    deg_chunks(rowT, colT, wT, half1 // c)
    piece = ept // 5
    for p in range(5):
      pltpu.sync_copy(row_h.at[pl.ds(obid * ept + p * piece, piece)], rowTo)
      pltpu.sync_copy(col_h.at[pl.ds(obid * ept + p * piece, piece)], colTo)
      pltpu.sync_copy(w_h.at[pl.ds(obid * ept + p * piece, piece)], wTo)
      deg_chunks(rowTo, colTo, wTo, piece // c)

    @pl.when(sid < ntiles_io)
    def _():
      pltpu.make_async_copy(y1_h.at[pl.ds(r0, rslice)],
                            y1_sh.at[pl.ds(r0, rslice)], gsem_a).wait()

    plsc.subcore_barrier()

    pltpu.sync_copy(deg_sh, dinv_v)

    def dinv_body(i, _):
      sl = pl.ds(i * L, L)
      dv = dinv_v[sl]
      dinv_v[sl] = jnp.where(dv > 0.0, _rsqrt_newton(jnp.maximum(dv, 1e-30)),
                             0.0)
      return 0

    lax.fori_loop(0, n // L, dinv_body, 0)

    slots = ((rows_a, colb_a, gsem_a, ssem_a), (rows_b, colb_b, gsem_b,
                                                ssem_b))

    def start_gather(i, slot):
      rows, _, sem, _ = slots[slot]
      pltpu.async_copy(y1_sh.at[rowT.at[pl.ds(i * c, c)]], rows, sem)

    def wait_gather(slot):
      rows, _, sem, _ = slots[slot]
      pltpu.make_async_copy(y1_h.at[pl.ds(0, c)], rows, sem).wait()

    def wait_scatter(slot):
      rows, colb, _, sem = slots[slot]
      pltpu.make_async_copy(rows, acc_sh.at[colb], sem).wait()

    def run_half(off, cnt):
      hchunk = cnt // c

      def process(i, slot):
        rows, colb, _, ssem = slots[slot]
        base = i * c
        for j in range(c // L):
          srcsl = pl.ds(base + j * L, L)
          rv, cv, wv = rowT[srcsl], colT[srcsl], wT[srcsl]
          dr = plsc.load_gather(dinv_v, [rv])
          dc = plsc.load_gather(dinv_v, [cv])
          weff = jnp.where(rv == cv, 0.0, wv)
          normT[srcsl] = -(dr * weff * dc)
          colb[pl.ds(j * L, L)] = cv
        wait_gather(slot)

        @pl.when(i + 1 < hchunk)
        def _():
          @pl.when(i >= 1)
          def _():
            wait_scatter(1 - slot)

          start_gather(i + 1, 1 - slot)

        for j in range(c // L):
          _scale_rows(rows, normT[pl.ds(base + j * L, L)], j, nslice)
        pltpu.async_copy(rows, acc_sh.at[colb], ssem, add=True)

      start_gather(0, 0)

      def seg_body(i2, _):
        for par in range(2):
          i = 2 * i2 + par

          @pl.when(i < hchunk)
          def _():
            process(i, par)

        return 0

      lax.fori_loop(0, (hchunk + 1) // 2, seg_body, 0)
      wait_scatter(0)
      wait_scatter(1)
      pltpu.sync_copy(normT.at[pl.ds(0, cnt)],
                      norm_h.at[pl.ds(gid * ept + off, cnt)])

    load_own(0, half0)
    run_half(0, half0)
    load_own(half0, half1)
    run_half(half0, half1)
    plsc.subcore_barrier()

    @pl.when(sid < ntiles_io)
    def _():
      pltpu.sync_copy(acc_sh.at[pl.ds(r0, rslice)],
                      s1_h.at[pl.ds(cid * n + r0, rslice)])

  return k(row, col, w, y1)


def _sc_layer2(n, e, d, c, row, col, norm, y2):
  ept = e // (NC * NS)
  nchunk = ept // c
  rslice = 1000
  ntiles_io = n // rslice
  nslice = d // L

  mesh = plsc.VectorSubcoreMesh(core_axis_name="c", subcore_axis_name="s")

  @functools.partial(
      pl.kernel,
      out_type=jax.ShapeDtypeStruct((2 * n, d), F32),
      mesh=mesh,
      compiler_params=pltpu.CompilerParams(needs_layout_passes=False, use_tc_tiling_on_sc=False),
      scratch_types=dict(
          acc_sh=pltpu.VMEM_SHARED((n, d), F32),
          y2_sh=pltpu.VMEM_SHARED((n, d), F32),
          rowT=pltpu.VMEM((ept,), I32),
          colT=pltpu.VMEM((ept,), I32),
          normT=pltpu.VMEM((ept,), F32),
          rows_a=pltpu.VMEM((c, d), F32),
          rows_b=pltpu.VMEM((c, d), F32),
          colb_a=pltpu.VMEM((c,), I32),
          colb_b=pltpu.VMEM((c,), I32),
          gsem_a=pltpu.SemaphoreType.DMA,
          gsem_b=pltpu.SemaphoreType.DMA,
          ssem_a=pltpu.SemaphoreType.DMA,
          ssem_b=pltpu.SemaphoreType.DMA,
      ),
  )
  def k(row_h, col_h, norm_h, y2_h, s2_h, *, acc_sh, y2_sh, rowT, colT,
        normT, rows_a, rows_b, colb_a, colb_b, gsem_a, gsem_b, ssem_a,
        ssem_b):
    cid = lax.axis_index("c")
    sid = lax.axis_index("s")
    gid = cid * NS + sid

    pltpu.async_copy(row_h.at[pl.ds(gid * ept, ept)], rowT, gsem_b)
    pltpu.async_copy(col_h.at[pl.ds(gid * ept, ept)], colT, gsem_b)
    pltpu.async_copy(norm_h.at[pl.ds(gid * ept, ept)], normT, gsem_b)

    r1 = sid * rslice

    @pl.when(sid < ntiles_io)
    def _():
      pltpu.async_copy(y2_h.at[pl.ds(r1, rslice)],
                       y2_sh.at[pl.ds(r1, rslice)], gsem_a)

    _zero_fill2(rows_a, c, d)

    r0 = sid * rslice
    nfull = rslice // c
    rem = rslice - nfull * c

    @pl.when(sid < ntiles_io)
    def _():
      for b in range(nfull):
        pltpu.sync_copy(rows_a, acc_sh.at[pl.ds(r0 + b * c, c)])
      if rem:
        pltpu.sync_copy(rows_a.at[pl.ds(0, rem)],
                        acc_sh.at[pl.ds(r0 + nfull * c, rem)])
      pltpu.make_async_copy(y2_h.at[pl.ds(r1, rslice)],
                            y2_sh.at[pl.ds(r1, rslice)], gsem_a).wait()

    pltpu.make_async_copy(row_h.at[pl.ds(gid * ept, ept)], rowT,
                          gsem_b).wait()
    pltpu.make_async_copy(col_h.at[pl.ds(gid * ept, ept)], colT,
                          gsem_b).wait()
    pltpu.make_async_copy(norm_h.at[pl.ds(gid * ept, ept)], normT,
                          gsem_b).wait()
    plsc.subcore_barrier()

    slots = ((rows_a, colb_a, gsem_a, ssem_a), (rows_b, colb_b, gsem_b,
                                                ssem_b))

    def start_gather(i, slot):
      rows, _, sem, _ = slots[slot]
      pltpu.async_copy(y2_sh.at[rowT.at[pl.ds(i * c, c)]], rows, sem)

    def wait_gather(slot):
      rows, _, sem, _ = slots[slot]
      pltpu.make_async_copy(y2_h.at[pl.ds(0, c)], rows, sem).wait()

    def wait_scatter(slot):
      rows, colb, _, sem = slots[slot]
      pltpu.make_async_copy(rows, acc_sh.at[colb], sem).wait()

    def process(i, slot):
      rows, colb, _, ssem = slots[slot]
      base = i * c
      for j in range(c // L):
        colb[pl.ds(j * L, L)] = colT[pl.ds(base + j * L, L)]
      wait_gather(slot)

      @pl.when(i + 1 < nchunk)
      def _():
        @pl.when(i >= 1)
        def _():
          wait_scatter(1 - slot)

        start_gather(i + 1, 1 - slot)

      for j in range(c // L):
        _scale_rows(rows, normT[pl.ds(base + j * L, L)], j, nslice)
      pltpu.async_copy(rows, acc_sh.at[colb], ssem, add=True)

    start_gather(0, 0)

    def seg_body(i2, _):
      for par in range(2):
        i = 2 * i2 + par

        @pl.when(i < nchunk)
        def _():
          process(i, par)

      return 0

    lax.fori_loop(0, (nchunk + 1) // 2, seg_body, 0)
    wait_scatter(0)
    wait_scatter(1)
    plsc.subcore_barrier()

    @pl.when(sid < ntiles_io)
    def _():
      pltpu.sync_copy(acc_sh.at[pl.ds(r0, rslice)],
                      s2_h.at[pl.ds(cid * n + r0, rslice)])

  return k(row, col, norm, y2)


def _dot_t(a, b):
  return lax.dot_general(a, b, (((1,), (1,)), ((), ())),
                         preferred_element_type=F32)


def _tc_matmul2(x, w1, w0, bn):
  n, kdim = x.shape
  m1 = w1.shape[0]
  m2 = w0.shape[0]

  def body(x_ref, w1_ref, w0_ref, o1_ref, o2_ref):
    xb = x_ref[...]
    o1_ref[...] = _dot_t(xb, w1_ref[...])
    o2_ref[...] = _dot_t(xb, w0_ref[...])

  return pl.pallas_call(
      body,
      grid=(n // bn,),
      in_specs=[
          pl.BlockSpec((bn, kdim), lambda i: (i, 0)),
          pl.BlockSpec((m1, kdim), lambda i: (0, 0)),
          pl.BlockSpec((m2, kdim), lambda i: (0, 0)),
      ],
      out_specs=[
          pl.BlockSpec((bn, m1), lambda i: (i, 0)),
          pl.BlockSpec((bn, m2), lambda i: (i, 0)),
      ],
      out_shape=[
          jax.ShapeDtypeStruct((n, m1), F32),
          jax.ShapeDtypeStruct((n, m2), F32),
      ],
  )(x, w1, w0)


def _tc_mid(xw0, s1, b1, w1_2, w0_2, d2, bn):
  n = xw0.shape[0]
  hid = s1.shape[1]
  ncls = w1_2.shape[0]

  def body(x_ref, s1a_ref, s1b_ref, b1_ref, w1_ref, w0_ref, o1_ref, o2_ref):
    h = x_ref[...] + s1a_ref[...] + s1b_ref[...] + b1_ref[...]
    h = jnp.maximum(h, 0.0)
    y2 = _dot_t(h, w1_ref[...])
    o1_ref[...] = jnp.concatenate(
        [y2, jnp.zeros((y2.shape[0], d2 - ncls), F32)], axis=1)
    o2_ref[...] = _dot_t(h, w0_ref[...])

  return pl.pallas_call(
      body,
      grid=(n // bn,),
      in_specs=[
          pl.BlockSpec((bn, hid), lambda i: (i, 0)),
          pl.BlockSpec((bn, hid), lambda i: (i, 0)),
          pl.BlockSpec((bn, hid), lambda i, n_blk=n // bn: (i + n_blk, 0)),
          pl.BlockSpec((1, hid), lambda i: (0, 0)),
          pl.BlockSpec((ncls, hid), lambda i: (0, 0)),
          pl.BlockSpec((ncls, hid), lambda i: (0, 0)),
      ],
      out_specs=[
          pl.BlockSpec((bn, d2), lambda i: (i, 0)),
          pl.BlockSpec((bn, ncls), lambda i: (i, 0)),
      ],
      out_shape=[
          jax.ShapeDtypeStruct((n, d2), F32),
          jax.ShapeDtypeStruct((n, ncls), F32),
      ],
  )(xw0, s1, s1, b1.reshape(1, hid), w1_2, w0_2)


def _tc_final(hw0, s2, b2, bn):
  n, ncls = hw0.shape
  d2 = s2.shape[1]

  def body(h_ref, s2a_ref, s2b_ref, b2_ref, o_ref):
    o_ref[...] = (h_ref[...] + s2a_ref[:, :ncls] + s2b_ref[:, :ncls]
                  + b2_ref[...])

  return pl.pallas_call(
      body,
      grid=(n // bn,),
      in_specs=[
          pl.BlockSpec((bn, ncls), lambda i: (i, 0)),
          pl.BlockSpec((bn, d2), lambda i: (i, 0)),
          pl.BlockSpec((bn, d2), lambda i, n_blk=n // bn: (i + n_blk, 0)),
          pl.BlockSpec((1, ncls), lambda i: (0, 0)),
      ],
      out_specs=pl.BlockSpec((bn, ncls), lambda i: (i, 0)),
      out_shape=jax.ShapeDtypeStruct((n, ncls), F32),
  )(hw0, s2, s2, b2.reshape(1, ncls))


@jax.jit
def kernel(x, edge_index, edge_weight, W0_1, W1_1, b1, W0_2, W1_2, b2):
  n, _ = x.shape
  e = edge_index.shape[1]
  hid = W0_1.shape[0]
  ncls = W0_2.shape[0]
  d2 = 48
  c = 80
  bn = 1000

  row = edge_index[0]
  col = edge_index[1]

  y1, xw0 = _tc_matmul2(x, W1_1, W0_1, bn)

  norm, s1 = _sc_layer1(n, e, hid, c, row, col, edge_weight, y1)

  y2, hw0 = _tc_mid(xw0, s1, b1, W1_2, W0_2, d2, bn)

  s2 = _sc_layer2(n, e, d2, c, row, col, norm, y2)

  return _tc_final(hw0, s2, b2, bn)
